# Initial kernel scaffold; baseline (speedup 1.0000x reference)
import functools

import jax
import jax.numpy as jnp
from jax import lax
from jax.experimental import pallas as pl
from jax.experimental.pallas import tpu as pltpu
from jax.experimental.pallas import tpu_sc as plsc

N = 10000
E = 320000
NTILES = 32
EW = E // NTILES
KC = 80
NCH = EW // KC
RPT = N // 16
BN = 1000
GRID = N // BN


def _make_sc(D):
    DP = D + 16
    TCH = D // 16
    mesh = plsc.VectorSubcoreMesh(core_axis_name="c", subcore_axis_name="s")
    out_type = [
        jax.ShapeDtypeStruct((2, N, DP), jnp.float32),
        jax.ShapeDtypeStruct((2, 16), jnp.float32),
    ]
    scratch = [
        pltpu.VMEM((N,), jnp.float32),
        pltpu.VMEM((N,), jnp.float32),
        pltpu.VMEM((NCH, KC), jnp.int32),
        pltpu.VMEM((NCH, KC), jnp.int32),
        pltpu.VMEM((NCH, KC), jnp.float32),
        pltpu.VMEM((NCH, KC), jnp.float32),
        pltpu.VMEM((KC,), jnp.float32),
        pltpu.VMEM((KC, D), jnp.float32),
        pltpu.VMEM((KC, DP), jnp.float32),
        pltpu.VMEM((16,), jnp.float32),
        pltpu.VMEM((16, 16), jnp.float32),
        pltpu.VMEM((16,), jnp.float32),
        pltpu.VMEM_SHARED((N, DP), jnp.float32),
        pltpu.VMEM_SHARED((16, 16), jnp.float32),
        pltpu.SemaphoreType.DMA,
    ]

    @functools.partial(pl.kernel, out_type=out_type, mesh=mesh,
                       scratch_types=scratch)
    def sck(h_hbm, as_hbm, ad_hbm, src_hbm, dst_hbm, ew_hbm, c_hbm,
            acc_hbm, m_hbm,
            asv, adv, srcv, dstv, ewv, av, pb, rowA, rowB, mxv, mx2v, cv,
            accsh, mxsh, sem):
        c = lax.axis_index("c")
        s = lax.axis_index("s")
        wid = s * 2 + c

        pltpu.sync_copy(as_hbm, asv)
        pltpu.sync_copy(ad_hbm, adv)
        pltpu.sync_copy(src_hbm.at[wid], srcv)
        pltpu.sync_copy(dst_hbm.at[wid], dstv)
        pltpu.sync_copy(ew_hbm.at[wid], ewv)
        pltpu.sync_copy(c_hbm, cv)
        ce = cv[0]

        def p1(j, mx):
            for k in range(KC // 16):
                sl = pl.ds(k * 16, 16)
                asg = plsc.load_gather(asv, [srcv[j, sl]])
                adg = plsc.load_gather(adv, [dstv[j, sl]])
                a = asg + adg + ce * ewv[j, sl]
                a = jnp.where(a >= 0.0, a, 0.2 * a)
                av[j, sl] = a
                mx = jnp.maximum(mx, a)
            return mx

        mx = lax.fori_loop(0, NCH, p1,
                           jnp.full((16,), -jnp.inf, jnp.float32))
        mxv[...] = mx
        pltpu.sync_copy(mxv, mxsh.at[s])
        plsc.subcore_barrier()
        pltpu.sync_copy(mxsh, mx2v)

        def pmax(k, mm):
            return jnp.maximum(mm, mx2v[k])

        mm = lax.fori_loop(0, 16, pmax,
                           jnp.full((16,), -jnp.inf, jnp.float32))
        msc = jnp.max(mm)

        @pl.when(s == 0)
        def _():
            mxv[...] = jnp.zeros((16,), jnp.float32) + msc
            pltpu.sync_copy(mxv, m_hbm.at[c])

        zero16 = jnp.zeros((16,), jnp.float32)

        def zrow(r, carry):
            for t in range(DP // 16):
                rowB[r, pl.ds(t * 16, 16)] = zero16
            return carry

        lax.fori_loop(0, KC, zrow, 0)
        base = s * RPT
        for i in range(RPT // KC):
            pltpu.sync_copy(rowB, accsh.at[pl.ds(base + i * KC, KC)])
        rem = RPT - (RPT // KC) * KC
        pltpu.sync_copy(rowB.at[pl.ds(0, rem)],
                        accsh.at[pl.ds(base + RPT - rem, rem)])
        plsc.subcore_barrier()

        lane0 = (lax.iota(jnp.int32, 16) == 0).astype(jnp.float32)

        def p3(j, carry):
            cp = pltpu.async_copy(h_hbm.at[srcv.at[j]], rowA, sem)
            for k in range(KC // 16):
                sl = pl.ds(k * 16, 16)
                pb[sl] = jnp.exp(av[j, sl] - msc)
            cp.wait()

            def rloop(r, rc):
                ps = pb[r]
                for t in range(TCH):
                    sl = pl.ds(t * 16, 16)
                    rowB[r, sl] = rowA[r, sl] * ps
                rowB[r, pl.ds(D, 16)] = lane0 * ps
                return rc

            lax.fori_loop(0, KC, rloop, 0)
            pltpu.sync_copy(rowB, accsh.at[dstv.at[j]], add=True)
            return carry

        lax.fori_loop(0, NCH, p3, 0)
        plsc.subcore_barrier()

        pltpu.sync_copy(accsh.at[pl.ds(base, RPT)],
                        acc_hbm.at[c].at[pl.ds(base, RPT)])

    return sck


def _dense1_body(x_ref, w_ref, as_ref, ad_ref, ew_ref,
                 h_ref, asp_ref, adp_ref, ews_ref):
    h = jnp.dot(x_ref[...], w_ref[...], preferred_element_type=jnp.float32)
    h_ref[...] = h
    asp_ref[...] = jnp.sum(h * as_ref[...], axis=1, keepdims=True)
    adp_ref[...] = jnp.sum(h * ad_ref[...], axis=1, keepdims=True)

    @pl.when(pl.program_id(0) == 0)
    def _():
        ews_ref[...] = jnp.zeros_like(ews_ref)

    ews_ref[...] = ews_ref[...] + jnp.sum(ew_ref[...])


def _dense1(x, W, a_s, a_d, ew):
    dout = W.shape[1]
    return pl.pallas_call(
        _dense1_body,
        grid=(GRID,),
        in_specs=[
            pl.BlockSpec((BN, W.shape[0]), lambda i: (i, 0)),
            pl.BlockSpec(W.shape, lambda i: (0, 0)),
            pl.BlockSpec((1, dout), lambda i: (0, 0)),
            pl.BlockSpec((1, dout), lambda i: (0, 0)),
            pl.BlockSpec((E // GRID, 1), lambda i: (i, 0)),
        ],
        out_specs=[
            pl.BlockSpec((BN, dout), lambda i: (i, 0)),
            pl.BlockSpec((BN, 1), lambda i: (i, 0)),
            pl.BlockSpec((BN, 1), lambda i: (i, 0)),
            pl.BlockSpec((1, 1), lambda i: (0, 0)),
        ],
        out_shape=[
            jax.ShapeDtypeStruct((N, dout), jnp.float32),
            jax.ShapeDtypeStruct((N, 1), jnp.float32),
            jax.ShapeDtypeStruct((N, 1), jnp.float32),
            jax.ShapeDtypeStruct((1, 1), jnp.float32),
        ],
        compiler_params=pltpu.CompilerParams(
            dimension_semantics=("arbitrary",)),
    )(x, W, a_s, a_d, ew)


def _combine_xin(Dp, acc0_ref, acc1_ref, m_ref, hp_ref, asp_ref, adp_ref,
                 bp_ref, lm_ref):
    m0 = jnp.max(m_ref[...][0:1, :])
    m1 = jnp.max(m_ref[...][1:2, :])
    g = jnp.maximum(m0, m1)
    f0 = jnp.exp(m0 - g)
    f1 = jnp.exp(m1 - g)
    al = asp_ref[...] + adp_ref[...] + lm_ref[...]
    al = jnp.where(al >= 0.0, al, 0.2 * al)
    ploop = jnp.exp(al - g)
    num = (acc0_ref[...][:, :Dp] * f0 + acc1_ref[...][:, :Dp] * f1
           + hp_ref[...] * ploop)
    den = (acc0_ref[...][:, Dp:Dp + 1] * f0
           + acc1_ref[...][:, Dp:Dp + 1] * f1 + ploop)
    xin = num / den + bp_ref[...]
    return jnp.where(xin >= 0.0, xin, 0.01 * xin)


def _make_combine_matmul(Dp, Dn):
    def body(acc0_ref, acc1_ref, m_ref, hp_ref, asp_ref, adp_ref, bp_ref,
             lm_ref, w_ref, as_ref, ad_ref, h_ref, aspo_ref, adpo_ref):
        xin = _combine_xin(Dp, acc0_ref, acc1_ref, m_ref, hp_ref, asp_ref,
                           adp_ref, bp_ref, lm_ref)
        h = jnp.dot(xin, w_ref[...], preferred_element_type=jnp.float32)
        h_ref[...] = h
        aspo_ref[...] = jnp.sum(h * as_ref[...], axis=1, keepdims=True)
        adpo_ref[...] = jnp.sum(h * ad_ref[...], axis=1, keepdims=True)

    def run(acc, m, hp, asp, adp, bp, lm, W, a_s, a_d):
        DPp = Dp + 16
        return pl.pallas_call(
            body,
            grid=(GRID,),
            in_specs=[
                pl.BlockSpec((BN, DPp), lambda i: (i, 0)),
                pl.BlockSpec((BN, DPp), lambda i: (i, 0)),
                pl.BlockSpec((2, 16), lambda i: (0, 0)),
                pl.BlockSpec((BN, Dp), lambda i: (i, 0)),
                pl.BlockSpec((BN, 1), lambda i: (i, 0)),
                pl.BlockSpec((BN, 1), lambda i: (i, 0)),
                pl.BlockSpec((1, Dp), lambda i: (0, 0)),
                pl.BlockSpec((1, 1), lambda i: (0, 0)),
                pl.BlockSpec((Dp, Dn), lambda i: (0, 0)),
                pl.BlockSpec((1, Dn), lambda i: (0, 0)),
                pl.BlockSpec((1, Dn), lambda i: (0, 0)),
            ],
            out_specs=[
                pl.BlockSpec((BN, Dn), lambda i: (i, 0)),
                pl.BlockSpec((BN, 1), lambda i: (i, 0)),
                pl.BlockSpec((BN, 1), lambda i: (i, 0)),
            ],
            out_shape=[
                jax.ShapeDtypeStruct((N, Dn), jnp.float32),
                jax.ShapeDtypeStruct((N, 1), jnp.float32),
                jax.ShapeDtypeStruct((N, 1), jnp.float32),
            ],
        )(acc[0], acc[1], m, hp, asp, adp, bp, lm, W, a_s, a_d)

    return run


def _make_combine_out(Dp):
    def body(acc0_ref, acc1_ref, m_ref, hp_ref, asp_ref, adp_ref, bp_ref,
             lm_ref, o_ref):
        o_ref[...] = _combine_xin(Dp, acc0_ref, acc1_ref, m_ref, hp_ref,
                                  asp_ref, adp_ref, bp_ref, lm_ref)

    def run(acc, m, hp, asp, adp, bp, lm):
        DPp = Dp + 16
        return pl.pallas_call(
            body,
            grid=(GRID,),
            in_specs=[
                pl.BlockSpec((BN, DPp), lambda i: (i, 0)),
                pl.BlockSpec((BN, DPp), lambda i: (i, 0)),
                pl.BlockSpec((2, 16), lambda i: (0, 0)),
                pl.BlockSpec((BN, Dp), lambda i: (i, 0)),
                pl.BlockSpec((BN, 1), lambda i: (i, 0)),
                pl.BlockSpec((BN, 1), lambda i: (i, 0)),
                pl.BlockSpec((1, Dp), lambda i: (0, 0)),
                pl.BlockSpec((1, 1), lambda i: (0, 0)),
            ],
            out_specs=pl.BlockSpec((BN, Dp), lambda i: (i, 0)),
            out_shape=jax.ShapeDtypeStruct((N, Dp), jnp.float32),
        )(acc[0], acc[1], m, hp, asp, adp, bp, lm)

    return run


_sc128 = _make_sc(128)
_sc64 = _make_sc(64)
_sc32 = _make_sc(32)
_comb12 = _make_combine_matmul(128, 64)
_comb23 = _make_combine_matmul(64, 32)
_comb3o = _make_combine_out(32)


def kernel(x, adj, edge_weight, W1, as1, ad1, We1, ae1, b1,
           W2, as2, ad2, We2, ae2, b2, W3, as3, ad3, We3, ae3, b3):
    src = adj[0].reshape(NTILES, NCH, KC)
    dst = adj[1].reshape(NTILES, NCH, KC)
    ew3 = edge_weight.reshape(NTILES, NCH, KC)

    c1 = jnp.sum(We1[0] * ae1)
    c2 = jnp.sum(We2[0] * ae2)
    c3 = jnp.sum(We3[0] * ae3)

    h1, asp1, adp1, ews = _dense1(x, W1, as1.reshape(1, -1),
                                  ad1.reshape(1, -1), edge_weight)
    acc1, m1 = _sc128(h1, asp1.reshape(N), adp1.reshape(N), src, dst, ew3,
                      jnp.full((16,), c1, jnp.float32))
    lm1 = (ews * (1.0 / E) * c1).reshape(1, 1)
    h2, asp2, adp2 = _comb12(acc1, m1, h1, asp1, adp1, b1.reshape(1, -1),
                             lm1, W2, as2.reshape(1, -1), ad2.reshape(1, -1))
    acc2, m2 = _sc64(h2, asp2.reshape(N), adp2.reshape(N), src, dst, ew3,
                     jnp.full((16,), c2, jnp.float32))
    lm2 = (ews * (1.0 / E) * c2).reshape(1, 1)
    h3, asp3, adp3 = _comb23(acc2, m2, h2, asp2, adp2, b2.reshape(1, -1),
                             lm2, W3, as3.reshape(1, -1), ad3.reshape(1, -1))
    acc3, m3 = _sc32(h3, asp3.reshape(N), adp3.reshape(N), src, dst, ew3,
                     jnp.full((16,), c3, jnp.float32))
    lm3 = (ews * (1.0 / E) * c3).reshape(1, 1)
    return _comb3o(acc3, m3, h3, asp3, adp3, b3.reshape(1, -1), lm3)


# speedup vs baseline: 22.5253x; 22.5253x over previous
import functools

import jax
import jax.numpy as jnp
from jax import lax
from jax.experimental import pallas as pl
from jax.experimental.pallas import tpu as pltpu
from jax.experimental.pallas import tpu_sc as plsc

N = 10000
E = 320000
NTILES = 32
EW = E // NTILES
KC = 80
NCH = EW // KC
RPT = N // 16
BN = 1000
GRID = N // BN


_SC_PARAMS = pltpu.CompilerParams(use_tc_tiling_on_sc=False,
                                  needs_layout_passes=False)
_MESH = plsc.VectorSubcoreMesh(core_axis_name="c", subcore_axis_name="s")


def _make_sc_alpha():
    out_type = [
        jax.ShapeDtypeStruct((NTILES, EW), jnp.float32),
        jax.ShapeDtypeStruct((2, 16), jnp.float32),
    ]
    scratch = [
        pltpu.VMEM((N,), jnp.float32),
        pltpu.VMEM((N,), jnp.float32),
        pltpu.VMEM((EW,), jnp.int32),
        pltpu.VMEM((EW,), jnp.int32),
        pltpu.VMEM((EW,), jnp.float32),
        pltpu.VMEM((EW,), jnp.float32),
        pltpu.VMEM((16,), jnp.float32),
        pltpu.VMEM((16, 16), jnp.float32),
        pltpu.VMEM((16,), jnp.float32),
        pltpu.VMEM_SHARED((16, 16), jnp.float32),
    ]

    @functools.partial(pl.kernel, out_type=out_type, mesh=_MESH,
                       scratch_types=scratch, compiler_params=_SC_PARAMS)
    def sck(as_hbm, ad_hbm, src_hbm, dst_hbm, ew_hbm, c_hbm,
            p_hbm, m_hbm,
            asv, adv, srcf, dstf, ewf, av, mxv, mx2v, cv, mxsh):
        c = lax.axis_index("c")
        s = lax.axis_index("s")
        wid = s * 2 + c

        pltpu.sync_copy(as_hbm, asv)
        pltpu.sync_copy(ad_hbm, adv)
        pltpu.sync_copy(src_hbm.at[wid], srcf)
        pltpu.sync_copy(dst_hbm.at[wid], dstf)
        pltpu.sync_copy(ew_hbm.at[wid], ewf)
        pltpu.sync_copy(c_hbm, cv)
        cev = cv[...]

        def p1(j, mx):
            sl = pl.ds(j * 16, 16)
            asg = plsc.load_gather(asv, [srcf[sl]])
            adg = plsc.load_gather(adv, [dstf[sl]])
            a = asg + adg + cev * ewf[sl]
            a = jnp.where(a >= 0.0, a, 0.2 * a)
            av[sl] = a
            return jnp.maximum(mx, a)

        mx = lax.fori_loop(0, EW // 16, p1,
                           jnp.full((16,), -jnp.inf, jnp.float32))
        mxv[...] = mx
        pltpu.sync_copy(mxv, mxsh.at[s])
        plsc.subcore_barrier()
        pltpu.sync_copy(mxsh, mx2v)

        def pmax(k, mm):
            return jnp.maximum(mm, mx2v[k])

        mm = lax.fori_loop(0, 16, pmax,
                           jnp.full((16,), -jnp.inf, jnp.float32))
        msc = jnp.max(mm)

        @pl.when(s == 0)
        def _():
            mxv[...] = jnp.zeros((16,), jnp.float32) + msc
            pltpu.sync_copy(mxv, m_hbm.at[c])

        def p2(j, carry):
            sl = pl.ds(j * 16, 16)
            av[sl] = jnp.exp(av[sl] - msc)
            return carry

        lax.fori_loop(0, EW // 16, p2, 0)
        pltpu.sync_copy(av, p_hbm.at[wid])

    return sck


def _make_sc_scatter(D):
    DP = D + 16
    TCH = D // 16
    out_type = [
        jax.ShapeDtypeStruct((2, N, DP), jnp.float32),
    ]
    scratch = [
        pltpu.VMEM((1, KC), jnp.int32),
        pltpu.VMEM((1, KC), jnp.int32),
        pltpu.VMEM((1, KC), jnp.float32),
        pltpu.VMEM((KC, D), jnp.float32),
        pltpu.VMEM((KC, DP), jnp.float32),
        pltpu.VMEM_SHARED((N, DP), jnp.float32),
        pltpu.SemaphoreType.DMA,
    ]

    @functools.partial(pl.kernel, out_type=out_type, mesh=_MESH,
                       scratch_types=scratch, compiler_params=_SC_PARAMS)
    def sck(h_hbm, src_hbm, dst_hbm, p_hbm,
            acc_hbm,
            srcb, dstb, pbuf, rowA, rowB, accsh, sem):
        c = lax.axis_index("c")
        s = lax.axis_index("s")
        wid = s * 2 + c

        zero16 = jnp.zeros((16,), jnp.float32)

        def zrow(r, carry):
            for t in range(DP // 16):
                rowB[r, pl.ds(t * 16, 16)] = zero16
            return carry

        lax.fori_loop(0, KC, zrow, 0)
        base = s * RPT
        for i in range(RPT // KC):
            pltpu.sync_copy(rowB, accsh.at[pl.ds(base + i * KC, KC)])
        rem = RPT - (RPT // KC) * KC
        pltpu.sync_copy(rowB.at[pl.ds(0, rem)],
                        accsh.at[pl.ds(base + RPT - rem, rem)])
        plsc.subcore_barrier()

        lane0 = (lax.iota(jnp.int32, 16) == 0).astype(jnp.float32)

        def p3(j, carry):
            pltpu.sync_copy(src_hbm.at[wid, j], srcb.at[0])
            pltpu.sync_copy(dst_hbm.at[wid, j], dstb.at[0])
            pltpu.sync_copy(p_hbm.at[wid, j], pbuf.at[0])
            pltpu.async_copy(h_hbm.at[srcb.at[0]], rowA, sem).wait()
            for k in range(KC // 16):
                pv = pbuf[0, pl.ds(k * 16, 16)]
                for kk in range(16):
                    r = k * 16 + kk
                    ps = pv[kk]
                    for t in range(TCH):
                        slt = pl.ds(t * 16, 16)
                        rowB[r, slt] = rowA[r, slt] * ps
                    rowB[r, pl.ds(D, 16)] = lane0 * ps
            pltpu.sync_copy(rowB, accsh.at[dstb.at[0]], add=True)
            return carry

        lax.fori_loop(0, NCH, p3, 0)
        plsc.subcore_barrier()

        pltpu.sync_copy(accsh.at[pl.ds(base, RPT)],
                        acc_hbm.at[c].at[pl.ds(base, RPT)])

    return sck


def _dense1_body(x_ref, w_ref, as_ref, ad_ref, ew_ref,
                 h_ref, asp_ref, adp_ref, ews_ref):
    h = jnp.dot(x_ref[...], w_ref[...], preferred_element_type=jnp.float32)
    h_ref[...] = h
    asp_ref[...] = jnp.sum(h * as_ref[...], axis=1, keepdims=True)
    adp_ref[...] = jnp.sum(h * ad_ref[...], axis=1, keepdims=True)

    @pl.when(pl.program_id(0) == 0)
    def _():
        ews_ref[...] = jnp.zeros_like(ews_ref)

    ews_ref[...] = ews_ref[...] + jnp.sum(ew_ref[...])


def _dense1(x, W, a_s, a_d, ew):
    dout = W.shape[1]
    return pl.pallas_call(
        _dense1_body,
        grid=(GRID,),
        in_specs=[
            pl.BlockSpec((BN, W.shape[0]), lambda i: (i, 0)),
            pl.BlockSpec(W.shape, lambda i: (0, 0)),
            pl.BlockSpec((1, dout), lambda i: (0, 0)),
            pl.BlockSpec((1, dout), lambda i: (0, 0)),
            pl.BlockSpec((E // GRID, 1), lambda i: (i, 0)),
        ],
        out_specs=[
            pl.BlockSpec((BN, dout), lambda i: (i, 0)),
            pl.BlockSpec((BN, 1), lambda i: (i, 0)),
            pl.BlockSpec((BN, 1), lambda i: (i, 0)),
            pl.BlockSpec((1, 1), lambda i: (0, 0)),
        ],
        out_shape=[
            jax.ShapeDtypeStruct((N, dout), jnp.float32),
            jax.ShapeDtypeStruct((N, 1), jnp.float32),
            jax.ShapeDtypeStruct((N, 1), jnp.float32),
            jax.ShapeDtypeStruct((1, 1), jnp.float32),
        ],
        compiler_params=pltpu.CompilerParams(
            dimension_semantics=("arbitrary",)),
    )(x, W, a_s, a_d, ew)


def _combine_xin(Dp, acc0_ref, acc1_ref, m_ref, hp_ref, asp_ref, adp_ref,
                 bp_ref, lm_ref):
    m0 = jnp.max(m_ref[...][0:1, :])
    m1 = jnp.max(m_ref[...][1:2, :])
    g = jnp.maximum(m0, m1)
    f0 = jnp.exp(m0 - g)
    f1 = jnp.exp(m1 - g)
    al = asp_ref[...] + adp_ref[...] + lm_ref[...]
    al = jnp.where(al >= 0.0, al, 0.2 * al)
    ploop = jnp.exp(al - g)
    num = (acc0_ref[...][:, :Dp] * f0 + acc1_ref[...][:, :Dp] * f1
           + hp_ref[...] * ploop)
    den = (acc0_ref[...][:, Dp:Dp + 1] * f0
           + acc1_ref[...][:, Dp:Dp + 1] * f1 + ploop)
    xin = num / den + bp_ref[...]
    return jnp.where(xin >= 0.0, xin, 0.01 * xin)


def _make_combine_matmul(Dp, Dn):
    def body(acc0_ref, acc1_ref, m_ref, hp_ref, asp_ref, adp_ref, bp_ref,
             lm_ref, w_ref, as_ref, ad_ref, h_ref, aspo_ref, adpo_ref):
        xin = _combine_xin(Dp, acc0_ref, acc1_ref, m_ref, hp_ref, asp_ref,
                           adp_ref, bp_ref, lm_ref)
        h = jnp.dot(xin, w_ref[...], preferred_element_type=jnp.float32)
        h_ref[...] = h
        aspo_ref[...] = jnp.sum(h * as_ref[...], axis=1, keepdims=True)
        adpo_ref[...] = jnp.sum(h * ad_ref[...], axis=1, keepdims=True)

    def run(acc, m, hp, asp, adp, bp, lm, W, a_s, a_d):
        DPp = Dp + 16
        return pl.pallas_call(
            body,
            grid=(GRID,),
            in_specs=[
                pl.BlockSpec((BN, DPp), lambda i: (i, 0)),
                pl.BlockSpec((BN, DPp), lambda i: (i, 0)),
                pl.BlockSpec((2, 16), lambda i: (0, 0)),
                pl.BlockSpec((BN, Dp), lambda i: (i, 0)),
                pl.BlockSpec((BN, 1), lambda i: (i, 0)),
                pl.BlockSpec((BN, 1), lambda i: (i, 0)),
                pl.BlockSpec((1, Dp), lambda i: (0, 0)),
                pl.BlockSpec((1, 1), lambda i: (0, 0)),
                pl.BlockSpec((Dp, Dn), lambda i: (0, 0)),
                pl.BlockSpec((1, Dn), lambda i: (0, 0)),
                pl.BlockSpec((1, Dn), lambda i: (0, 0)),
            ],
            out_specs=[
                pl.BlockSpec((BN, Dn), lambda i: (i, 0)),
                pl.BlockSpec((BN, 1), lambda i: (i, 0)),
                pl.BlockSpec((BN, 1), lambda i: (i, 0)),
            ],
            out_shape=[
                jax.ShapeDtypeStruct((N, Dn), jnp.float32),
                jax.ShapeDtypeStruct((N, 1), jnp.float32),
                jax.ShapeDtypeStruct((N, 1), jnp.float32),
            ],
        )(acc[0], acc[1], m, hp, asp, adp, bp, lm, W, a_s, a_d)

    return run


def _make_combine_out(Dp):
    def body(acc0_ref, acc1_ref, m_ref, hp_ref, asp_ref, adp_ref, bp_ref,
             lm_ref, o_ref):
        o_ref[...] = _combine_xin(Dp, acc0_ref, acc1_ref, m_ref, hp_ref,
                                  asp_ref, adp_ref, bp_ref, lm_ref)

    def run(acc, m, hp, asp, adp, bp, lm):
        DPp = Dp + 16
        return pl.pallas_call(
            body,
            grid=(GRID,),
            in_specs=[
                pl.BlockSpec((BN, DPp), lambda i: (i, 0)),
                pl.BlockSpec((BN, DPp), lambda i: (i, 0)),
                pl.BlockSpec((2, 16), lambda i: (0, 0)),
                pl.BlockSpec((BN, Dp), lambda i: (i, 0)),
                pl.BlockSpec((BN, 1), lambda i: (i, 0)),
                pl.BlockSpec((BN, 1), lambda i: (i, 0)),
                pl.BlockSpec((1, Dp), lambda i: (0, 0)),
                pl.BlockSpec((1, 1), lambda i: (0, 0)),
            ],
            out_specs=pl.BlockSpec((BN, Dp), lambda i: (i, 0)),
            out_shape=jax.ShapeDtypeStruct((N, Dp), jnp.float32),
        )(acc[0], acc[1], m, hp, asp, adp, bp, lm)

    return run


_sc_alpha = _make_sc_alpha()
_sc_scat128 = _make_sc_scatter(128)
_sc_scat64 = _make_sc_scatter(64)
_sc_scat32 = _make_sc_scatter(32)
_comb12 = _make_combine_matmul(128, 64)
_comb23 = _make_combine_matmul(64, 32)
_comb3o = _make_combine_out(32)


def kernel(x, adj, edge_weight, W1, as1, ad1, We1, ae1, b1,
           W2, as2, ad2, We2, ae2, b2, W3, as3, ad3, We3, ae3, b3):
    srcf = adj[0].reshape(NTILES, EW)
    dstf = adj[1].reshape(NTILES, EW)
    ewf = edge_weight.reshape(NTILES, EW)
    src3 = adj[0].reshape(NTILES, NCH, KC)
    dst3 = adj[1].reshape(NTILES, NCH, KC)

    c1 = jnp.sum(We1[0] * ae1)
    c2 = jnp.sum(We2[0] * ae2)
    c3 = jnp.sum(We3[0] * ae3)

    def gat_sc(h, asp, adp, cval, scat):
        p, m = _sc_alpha(asp.reshape(N), adp.reshape(N), srcf, dstf, ewf,
                         jnp.full((16,), cval, jnp.float32))
        acc = scat(h, src3, dst3, p.reshape(NTILES, NCH, KC))[0]
        return acc, m

    h1, asp1, adp1, ews = _dense1(x, W1, as1.reshape(1, -1),
                                  ad1.reshape(1, -1), edge_weight)
    acc1, m1 = gat_sc(h1, asp1, adp1, c1, _sc_scat128)
    lm1 = (ews * (1.0 / E) * c1).reshape(1, 1)
    h2, asp2, adp2 = _comb12(acc1, m1, h1, asp1, adp1, b1.reshape(1, -1),
                             lm1, W2, as2.reshape(1, -1), ad2.reshape(1, -1))
    acc2, m2 = gat_sc(h2, asp2, adp2, c2, _sc_scat64)
    lm2 = (ews * (1.0 / E) * c2).reshape(1, 1)
    h3, asp3, adp3 = _comb23(acc2, m2, h2, asp2, adp2, b2.reshape(1, -1),
                             lm2, W3, as3.reshape(1, -1), ad3.reshape(1, -1))
    acc3, m3 = gat_sc(h3, asp3, adp3, c3, _sc_scat32)
    lm3 = (ews * (1.0 / E) * c3).reshape(1, 1)
    return _comb3o(acc3, m3, h3, asp3, adp3, b3.reshape(1, -1), lm3)


# speedup vs baseline: 36.4772x; 1.6194x over previous
import functools

import jax
import jax.numpy as jnp
from jax import lax
from jax.experimental import pallas as pl
from jax.experimental.pallas import tpu as pltpu
from jax.experimental.pallas import tpu_sc as plsc

N = 10000
E = 320000
NTILES = 32
EW = E // NTILES
KC = 80
NCH = EW // KC
RPT = N // 16
BN = 1000
GRID = N // BN


_SC_PARAMS = pltpu.CompilerParams(use_tc_tiling_on_sc=False,
                                  needs_layout_passes=False)
_MESH = plsc.VectorSubcoreMesh(core_axis_name="c", subcore_axis_name="s")


def _make_sc_alpha():
    out_type = [
        jax.ShapeDtypeStruct((NTILES, EW), jnp.float32),
        jax.ShapeDtypeStruct((2, 16), jnp.float32),
    ]
    scratch = [
        pltpu.VMEM((N,), jnp.float32),
        pltpu.VMEM((N,), jnp.float32),
        pltpu.VMEM((EW,), jnp.int32),
        pltpu.VMEM((EW,), jnp.int32),
        pltpu.VMEM((EW,), jnp.float32),
        pltpu.VMEM((EW,), jnp.float32),
        pltpu.VMEM((16,), jnp.float32),
        pltpu.VMEM((16, 16), jnp.float32),
        pltpu.VMEM((16,), jnp.float32),
        pltpu.VMEM_SHARED((16, 16), jnp.float32),
    ]

    @functools.partial(pl.kernel, out_type=out_type, mesh=_MESH,
                       scratch_types=scratch, compiler_params=_SC_PARAMS)
    def sck(as_hbm, ad_hbm, src_hbm, dst_hbm, ew_hbm, c_hbm,
            p_hbm, m_hbm,
            asv, adv, srcf, dstf, ewf, av, mxv, mx2v, cv, mxsh):
        c = lax.axis_index("c")
        s = lax.axis_index("s")
        wid = s * 2 + c

        pltpu.sync_copy(as_hbm, asv)
        pltpu.sync_copy(ad_hbm, adv)
        pltpu.sync_copy(src_hbm.at[wid], srcf)
        pltpu.sync_copy(dst_hbm.at[wid], dstf)
        pltpu.sync_copy(ew_hbm.at[wid], ewf)
        pltpu.sync_copy(c_hbm, cv)
        cev = cv[...]

        def p1(j, mx):
            sl = pl.ds(j * 16, 16)
            asg = plsc.load_gather(asv, [srcf[sl]])
            adg = plsc.load_gather(adv, [dstf[sl]])
            a = asg + adg + cev * ewf[sl]
            a = jnp.where(a >= 0.0, a, 0.2 * a)
            av[sl] = a
            return jnp.maximum(mx, a)

        mx = lax.fori_loop(0, EW // 16, p1,
                           jnp.full((16,), -jnp.inf, jnp.float32))
        mxv[...] = mx
        pltpu.sync_copy(mxv, mxsh.at[s])
        plsc.subcore_barrier()
        pltpu.sync_copy(mxsh, mx2v)

        def pmax(k, mm):
            return jnp.maximum(mm, mx2v[k])

        mm = lax.fori_loop(0, 16, pmax,
                           jnp.full((16,), -jnp.inf, jnp.float32))
        msc = jnp.max(mm)

        @pl.when(s == 0)
        def _():
            mxv[...] = jnp.zeros((16,), jnp.float32) + msc
            pltpu.sync_copy(mxv, m_hbm.at[c])

        def p2(j, carry):
            sl = pl.ds(j * 16, 16)
            av[sl] = jnp.exp(av[sl] - msc)
            return carry

        lax.fori_loop(0, EW // 16, p2, 0)
        pltpu.sync_copy(av, p_hbm.at[wid])

    return sck


def _make_sc_scatter(D):
    TCH = D // 16
    out_type = [
        jax.ShapeDtypeStruct((2, N, D), jnp.float32),
        jax.ShapeDtypeStruct((2, N, 16), jnp.float32),
    ]
    scratch = [
        pltpu.VMEM((2, KC), jnp.int32),
        pltpu.VMEM((2, KC), jnp.int32),
        pltpu.VMEM((2, KC), jnp.float32),
        pltpu.VMEM((2, KC), jnp.int32),
        pltpu.VMEM((2, KC, D), jnp.float32),
        pltpu.VMEM((2, KC, 16), jnp.float32),
        pltpu.VMEM_SHARED((N, D), jnp.float32),
        pltpu.VMEM_SHARED((N, 16), jnp.float32),
        pltpu.SemaphoreType.DMA((2,)),
        pltpu.SemaphoreType.DMA((2,)),
        pltpu.SemaphoreType.DMA((2,)),
    ]

    @functools.partial(pl.kernel, out_type=out_type, mesh=_MESH,
                       scratch_types=scratch, compiler_params=_SC_PARAMS)
    def sck(h_hbm, src_hbm, dst_hbm, p_hbm,
            acc_hbm, den_hbm,
            srcb, dstb, pbuf, dsts, rowA, denb, accsh, densh,
            semi, semg, sems):
        c = lax.axis_index("c")
        s = lax.axis_index("s")
        wid = s * 2 + c
        zero16 = jnp.zeros((16,), jnp.float32)
        lane0 = (lax.iota(jnp.int32, 16) == 0).astype(jnp.float32)

        def issue_idx(j, par):
            pltpu.async_copy(src_hbm.at[wid, j], srcb.at[par], semi.at[par])
            pltpu.async_copy(dst_hbm.at[wid, j], dstb.at[par], semi.at[par])
            pltpu.async_copy(p_hbm.at[wid, j], pbuf.at[par], semi.at[par])

        def drain_idx(par):
            pltpu.make_async_copy(src_hbm.at[wid, 0], srcb.at[par],
                                  semi.at[par]).wait()
            pltpu.make_async_copy(dst_hbm.at[wid, 0], dstb.at[par],
                                  semi.at[par]).wait()
            pltpu.make_async_copy(p_hbm.at[wid, 0], pbuf.at[par],
                                  semi.at[par]).wait()

        def issue_gather(par):
            pltpu.async_copy(h_hbm.at[srcb.at[par]], rowA.at[par],
                             semg.at[par])

        def drain_gather(par):
            pltpu.make_async_copy(h_hbm.at[pl.ds(0, KC)], rowA.at[par],
                                  semg.at[par]).wait()

        def issue_scatter(par):
            pltpu.async_copy(rowA.at[par], accsh.at[dsts.at[par]],
                             sems.at[par], add=True)
            pltpu.async_copy(denb.at[par], densh.at[dsts.at[par]],
                             sems.at[par], add=True)

        def drain_scatter(par):
            pltpu.make_async_copy(rowA.at[par], accsh.at[pl.ds(0, KC)],
                                  sems.at[par]).wait()
            pltpu.make_async_copy(denb.at[par], densh.at[pl.ds(0, KC)],
                                  sems.at[par]).wait()

        def zrow(r, carry):
            for t in range(TCH):
                rowA[0, r, pl.ds(t * 16, 16)] = zero16
            denb[0, r, pl.ds(0, 16)] = zero16
            return carry

        lax.fori_loop(0, KC, zrow, 0)
        base = s * RPT
        for i in range(RPT // KC):
            pltpu.sync_copy(rowA.at[0], accsh.at[pl.ds(base + i * KC, KC)])
            pltpu.sync_copy(denb.at[0], densh.at[pl.ds(base + i * KC, KC)])
        rem = RPT - (RPT // KC) * KC
        lastz = pl.ds(base + RPT - rem, rem)
        pltpu.sync_copy(rowA.at[0].at[pl.ds(0, rem)], accsh.at[lastz])
        pltpu.sync_copy(denb.at[0].at[pl.ds(0, rem)], densh.at[lastz])
        plsc.subcore_barrier()

        issue_idx(0, 0)
        drain_idx(0)
        issue_gather(0)
        issue_idx(1, 1)

        def p3(j, carry):
            p = lax.rem(j, 2)
            q = 1 - p

            @pl.when(j + 1 < NCH)
            def _():
                drain_idx(q)

                @pl.when(j >= 1)
                def _():
                    drain_scatter(q)

                issue_gather(q)

            for k in range(KC // 16):
                sl = pl.ds(k * 16, 16)
                dsts[p, sl] = dstb[p, sl]
            drain_gather(p)
            for k in range(KC // 16):
                pv = pbuf[p, pl.ds(k * 16, 16)]
                for kk in range(16):
                    r = k * 16 + kk
                    ps = pv[kk]
                    for t in range(TCH):
                        slt = pl.ds(t * 16, 16)
                        rowA[p, r, slt] = rowA[p, r, slt] * ps
                    denb[p, r, pl.ds(0, 16)] = lane0 * ps

            @pl.when(j + 2 < NCH)
            def _():
                issue_idx(j + 2, p)

            issue_scatter(p)
            return carry

        lax.fori_loop(0, NCH, p3, 0)
        drain_scatter(0)
        drain_scatter(1)
        plsc.subcore_barrier()

        pltpu.sync_copy(accsh.at[pl.ds(base, RPT)],
                        acc_hbm.at[c].at[pl.ds(base, RPT)])
        pltpu.sync_copy(densh.at[pl.ds(base, RPT)],
                        den_hbm.at[c].at[pl.ds(base, RPT)])

    return sck


def _dense1_body(x_ref, w_ref, as_ref, ad_ref, ew_ref,
                 h_ref, asp_ref, adp_ref, ews_ref):
    h = jnp.dot(x_ref[...], w_ref[...], preferred_element_type=jnp.float32)
    h_ref[...] = h
    asp_ref[...] = jnp.sum(h * as_ref[...], axis=1, keepdims=True)
    adp_ref[...] = jnp.sum(h * ad_ref[...], axis=1, keepdims=True)

    @pl.when(pl.program_id(0) == 0)
    def _():
        ews_ref[...] = jnp.zeros_like(ews_ref)

    ews_ref[...] = ews_ref[...] + jnp.sum(ew_ref[...])


def _dense1(x, W, a_s, a_d, ew):
    dout = W.shape[1]
    return pl.pallas_call(
        _dense1_body,
        grid=(GRID,),
        in_specs=[
            pl.BlockSpec((BN, W.shape[0]), lambda i: (i, 0)),
            pl.BlockSpec(W.shape, lambda i: (0, 0)),
            pl.BlockSpec((1, dout), lambda i: (0, 0)),
            pl.BlockSpec((1, dout), lambda i: (0, 0)),
            pl.BlockSpec((E // GRID, 1), lambda i: (i, 0)),
        ],
        out_specs=[
            pl.BlockSpec((BN, dout), lambda i: (i, 0)),
            pl.BlockSpec((BN, 1), lambda i: (i, 0)),
            pl.BlockSpec((BN, 1), lambda i: (i, 0)),
            pl.BlockSpec((1, 1), lambda i: (0, 0)),
        ],
        out_shape=[
            jax.ShapeDtypeStruct((N, dout), jnp.float32),
            jax.ShapeDtypeStruct((N, 1), jnp.float32),
            jax.ShapeDtypeStruct((N, 1), jnp.float32),
            jax.ShapeDtypeStruct((1, 1), jnp.float32),
        ],
        compiler_params=pltpu.CompilerParams(
            dimension_semantics=("arbitrary",)),
    )(x, W, a_s, a_d, ew)


def _combine_xin(Dp, acc0_ref, acc1_ref, den0_ref, den1_ref, m_ref, hp_ref,
                 asp_ref, adp_ref, bp_ref, lm_ref):
    m0 = jnp.max(m_ref[...][0:1, :])
    m1 = jnp.max(m_ref[...][1:2, :])
    g = jnp.maximum(m0, m1)
    f0 = jnp.exp(m0 - g)
    f1 = jnp.exp(m1 - g)
    al = asp_ref[...] + adp_ref[...] + lm_ref[...]
    al = jnp.where(al >= 0.0, al, 0.2 * al)
    ploop = jnp.exp(al - g)
    num = acc0_ref[...] * f0 + acc1_ref[...] * f1 + hp_ref[...] * ploop
    den = (den0_ref[...][:, 0:1] * f0 + den1_ref[...][:, 0:1] * f1 + ploop)
    xin = num / den + bp_ref[...]
    return jnp.where(xin >= 0.0, xin, 0.01 * xin)


def _make_combine_matmul(Dp, Dn):
    def body(acc0_ref, acc1_ref, den0_ref, den1_ref, m_ref, hp_ref, asp_ref,
             adp_ref, bp_ref, lm_ref, w_ref, as_ref, ad_ref,
             h_ref, aspo_ref, adpo_ref):
        xin = _combine_xin(Dp, acc0_ref, acc1_ref, den0_ref, den1_ref, m_ref,
                           hp_ref, asp_ref, adp_ref, bp_ref, lm_ref)
        h = jnp.dot(xin, w_ref[...], preferred_element_type=jnp.float32)
        h_ref[...] = h
        aspo_ref[...] = jnp.sum(h * as_ref[...], axis=1, keepdims=True)
        adpo_ref[...] = jnp.sum(h * ad_ref[...], axis=1, keepdims=True)

    def run(acc, den, m, hp, asp, adp, bp, lm, W, a_s, a_d):
        return pl.pallas_call(
            body,
            grid=(GRID,),
            in_specs=[
                pl.BlockSpec((BN, Dp), lambda i: (i, 0)),
                pl.BlockSpec((BN, Dp), lambda i: (i, 0)),
                pl.BlockSpec((BN, 16), lambda i: (i, 0)),
                pl.BlockSpec((BN, 16), lambda i: (i, 0)),
                pl.BlockSpec((2, 16), lambda i: (0, 0)),
                pl.BlockSpec((BN, Dp), lambda i: (i, 0)),
                pl.BlockSpec((BN, 1), lambda i: (i, 0)),
                pl.BlockSpec((BN, 1), lambda i: (i, 0)),
                pl.BlockSpec((1, Dp), lambda i: (0, 0)),
                pl.BlockSpec((1, 1), lambda i: (0, 0)),
                pl.BlockSpec((Dp, Dn), lambda i: (0, 0)),
                pl.BlockSpec((1, Dn), lambda i: (0, 0)),
                pl.BlockSpec((1, Dn), lambda i: (0, 0)),
            ],
            out_specs=[
                pl.BlockSpec((BN, Dn), lambda i: (i, 0)),
                pl.BlockSpec((BN, 1), lambda i: (i, 0)),
                pl.BlockSpec((BN, 1), lambda i: (i, 0)),
            ],
            out_shape=[
                jax.ShapeDtypeStruct((N, Dn), jnp.float32),
                jax.ShapeDtypeStruct((N, 1), jnp.float32),
                jax.ShapeDtypeStruct((N, 1), jnp.float32),
            ],
        )(acc[0], acc[1], den[0], den[1], m, hp, asp, adp, bp, lm,
          W, a_s, a_d)

    return run


def _make_combine_out(Dp):
    def body(acc0_ref, acc1_ref, den0_ref, den1_ref, m_ref, hp_ref, asp_ref,
             adp_ref, bp_ref, lm_ref, o_ref):
        o_ref[...] = _combine_xin(Dp, acc0_ref, acc1_ref, den0_ref, den1_ref,
                                  m_ref, hp_ref, asp_ref, adp_ref, bp_ref,
                                  lm_ref)

    def run(acc, den, m, hp, asp, adp, bp, lm):
        return pl.pallas_call(
            body,
            grid=(GRID,),
            in_specs=[
                pl.BlockSpec((BN, Dp), lambda i: (i, 0)),
                pl.BlockSpec((BN, Dp), lambda i: (i, 0)),
                pl.BlockSpec((BN, 16), lambda i: (i, 0)),
                pl.BlockSpec((BN, 16), lambda i: (i, 0)),
                pl.BlockSpec((2, 16), lambda i: (0, 0)),
                pl.BlockSpec((BN, Dp), lambda i: (i, 0)),
                pl.BlockSpec((BN, 1), lambda i: (i, 0)),
                pl.BlockSpec((BN, 1), lambda i: (i, 0)),
                pl.BlockSpec((1, Dp), lambda i: (0, 0)),
                pl.BlockSpec((1, 1), lambda i: (0, 0)),
            ],
            out_specs=pl.BlockSpec((BN, Dp), lambda i: (i, 0)),
            out_shape=jax.ShapeDtypeStruct((N, Dp), jnp.float32),
        )(acc[0], acc[1], den[0], den[1], m, hp, asp, adp, bp, lm)

    return run


_sc_alpha = _make_sc_alpha()
_sc_scat128 = _make_sc_scatter(128)
_sc_scat64 = _make_sc_scatter(64)
_sc_scat32 = _make_sc_scatter(32)
_comb12 = _make_combine_matmul(128, 64)
_comb23 = _make_combine_matmul(64, 32)
_comb3o = _make_combine_out(32)


def kernel(x, adj, edge_weight, W1, as1, ad1, We1, ae1, b1,
           W2, as2, ad2, We2, ae2, b2, W3, as3, ad3, We3, ae3, b3):
    srcf = adj[0].reshape(NTILES, EW)
    dstf = adj[1].reshape(NTILES, EW)
    ewf = edge_weight.reshape(NTILES, EW)
    src3 = adj[0].reshape(NTILES, NCH, KC)
    dst3 = adj[1].reshape(NTILES, NCH, KC)

    c1 = jnp.sum(We1[0] * ae1)
    c2 = jnp.sum(We2[0] * ae2)
    c3 = jnp.sum(We3[0] * ae3)

    def gat_sc(h, asp, adp, cval, scat):
        p, m = _sc_alpha(asp.reshape(N), adp.reshape(N), srcf, dstf, ewf,
                         jnp.full((16,), cval, jnp.float32))
        acc, den = scat(h, src3, dst3, p.reshape(NTILES, NCH, KC))
        return acc, den, m

    h1, asp1, adp1, ews = _dense1(x, W1, as1.reshape(1, -1),
                                  ad1.reshape(1, -1), edge_weight)
    acc1, den1, m1 = gat_sc(h1, asp1, adp1, c1, _sc_scat128)
    lm1 = (ews * (1.0 / E) * c1).reshape(1, 1)
    h2, asp2, adp2 = _comb12(acc1, den1, m1, h1, asp1, adp1,
                             b1.reshape(1, -1), lm1, W2,
                             as2.reshape(1, -1), ad2.reshape(1, -1))
    acc2, den2, m2 = gat_sc(h2, asp2, adp2, c2, _sc_scat64)
    lm2 = (ews * (1.0 / E) * c2).reshape(1, 1)
    h3, asp3, adp3 = _comb23(acc2, den2, m2, h2, asp2, adp2,
                             b2.reshape(1, -1), lm2, W3,
                             as3.reshape(1, -1), ad3.reshape(1, -1))
    acc3, den3, m3 = gat_sc(h3, asp3, adp3, c3, _sc_scat32)
    lm3 = (ews * (1.0 / E) * c3).reshape(1, 1)
    return _comb3o(acc3, den3, m3, h3, asp3, adp3, b3.reshape(1, -1), lm3)


# speedup vs baseline: 43.2051x; 1.1844x over previous
import functools

import jax
import jax.numpy as jnp
from jax import lax
from jax.experimental import pallas as pl
from jax.experimental.pallas import tpu as pltpu
from jax.experimental.pallas import tpu_sc as plsc

N = 10000
E = 320000
NTILES = 32
EW = E // NTILES
KC = 80
NCH = EW // KC
RPT = N // 16
BN = 1000
GRID = N // BN


_SC_PARAMS = pltpu.CompilerParams(use_tc_tiling_on_sc=False,
                                  needs_layout_passes=False)
_MESH = plsc.VectorSubcoreMesh(core_axis_name="c", subcore_axis_name="s")


def _make_sc_alpha():
    out_type = [
        jax.ShapeDtypeStruct((NTILES, EW), jnp.float32),
        jax.ShapeDtypeStruct((2, 16), jnp.float32),
    ]
    scratch = [
        pltpu.VMEM((N,), jnp.float32),
        pltpu.VMEM((N,), jnp.float32),
        pltpu.VMEM((EW,), jnp.int32),
        pltpu.VMEM((EW,), jnp.int32),
        pltpu.VMEM((EW,), jnp.float32),
        pltpu.VMEM((EW,), jnp.float32),
        pltpu.VMEM((16,), jnp.float32),
        pltpu.VMEM((16, 16), jnp.float32),
        pltpu.VMEM((16,), jnp.float32),
        pltpu.VMEM_SHARED((16, 16), jnp.float32),
    ]

    @functools.partial(pl.kernel, out_type=out_type, mesh=_MESH,
                       scratch_types=scratch, compiler_params=_SC_PARAMS)
    def sck(as_hbm, ad_hbm, src_hbm, dst_hbm, ew_hbm, c_hbm,
            p_hbm, m_hbm,
            asv, adv, srcf, dstf, ewf, av, mxv, mx2v, cv, mxsh):
        c = lax.axis_index("c")
        s = lax.axis_index("s")
        wid = s * 2 + c

        pltpu.sync_copy(as_hbm, asv)
        pltpu.sync_copy(ad_hbm, adv)
        pltpu.sync_copy(src_hbm.at[wid], srcf)
        pltpu.sync_copy(dst_hbm.at[wid], dstf)
        pltpu.sync_copy(ew_hbm.at[wid], ewf)
        pltpu.sync_copy(c_hbm, cv)
        cev = cv[...]

        def p1(j, mx):
            sl = pl.ds(j * 16, 16)
            asg = plsc.load_gather(asv, [srcf[sl]])
            adg = plsc.load_gather(adv, [dstf[sl]])
            a = asg + adg + cev * ewf[sl]
            a = jnp.where(a >= 0.0, a, 0.2 * a)
            av[sl] = a
            return jnp.maximum(mx, a)

        mx = lax.fori_loop(0, EW // 16, p1,
                           jnp.full((16,), -jnp.inf, jnp.float32))
        mxv[...] = mx
        pltpu.sync_copy(mxv, mxsh.at[s])
        plsc.subcore_barrier()
        pltpu.sync_copy(mxsh, mx2v)

        def pmax(k, mm):
            return jnp.maximum(mm, mx2v[k])

        mm = lax.fori_loop(0, 16, pmax,
                           jnp.full((16,), -jnp.inf, jnp.float32))
        msc = jnp.max(mm)

        @pl.when(s == 0)
        def _():
            mxv[...] = jnp.zeros((16,), jnp.float32) + msc
            pltpu.sync_copy(mxv, m_hbm.at[c])

        def p2(j, carry):
            sl = pl.ds(j * 16, 16)
            av[sl] = jnp.exp(av[sl] - msc)
            return carry

        lax.fori_loop(0, EW // 16, p2, 0)
        pltpu.sync_copy(av, p_hbm.at[wid])

    return sck


def _make_sc_scatter(D, KCL):
    TCH = D // 16
    NCHL = EW // KCL
    out_type = [
        jax.ShapeDtypeStruct((2, N, D), jnp.float32),
        jax.ShapeDtypeStruct((2, N, 16), jnp.float32),
    ]
    scratch = [
        pltpu.VMEM((3, KCL), jnp.int32),
        pltpu.VMEM((3, KCL), jnp.int32),
        pltpu.VMEM((3, KCL), jnp.float32),
        pltpu.VMEM((3, KCL), jnp.int32),
        pltpu.VMEM((3, KCL, D), jnp.float32),
        pltpu.VMEM((3, KCL, 16), jnp.float32),
        pltpu.VMEM_SHARED((N, D), jnp.float32),
        pltpu.VMEM_SHARED((N, 16), jnp.float32),
        pltpu.SemaphoreType.DMA((3,)),
        pltpu.SemaphoreType.DMA((3,)),
        pltpu.SemaphoreType.DMA((3,)),
    ]

    @functools.partial(pl.kernel, out_type=out_type, mesh=_MESH,
                       scratch_types=scratch, compiler_params=_SC_PARAMS)
    def sck(h_hbm, src_hbm, dst_hbm, p_hbm,
            acc_hbm, den_hbm,
            srcb, dstb, pbuf, dsts, rowA, denb, accsh, densh,
            semi, semg, sems):
        c = lax.axis_index("c")
        s = lax.axis_index("s")
        wid = s * 2 + c
        zero16 = jnp.zeros((16,), jnp.float32)
        lane0 = (lax.iota(jnp.int32, 16) == 0).astype(jnp.float32)

        def issue_idx(j, par):
            pltpu.async_copy(src_hbm.at[wid, j], srcb.at[par], semi.at[par])
            pltpu.async_copy(dst_hbm.at[wid, j], dstb.at[par], semi.at[par])
            pltpu.async_copy(p_hbm.at[wid, j], pbuf.at[par], semi.at[par])

        def drain_idx(par):
            pltpu.make_async_copy(src_hbm.at[wid, 0], srcb.at[par],
                                  semi.at[par]).wait()
            pltpu.make_async_copy(dst_hbm.at[wid, 0], dstb.at[par],
                                  semi.at[par]).wait()
            pltpu.make_async_copy(p_hbm.at[wid, 0], pbuf.at[par],
                                  semi.at[par]).wait()

        def issue_gather(par):
            pltpu.async_copy(h_hbm.at[srcb.at[par]], rowA.at[par],
                             semg.at[par])

        def drain_gather(par):
            pltpu.make_async_copy(h_hbm.at[pl.ds(0, KCL)], rowA.at[par],
                                  semg.at[par]).wait()

        def issue_scatter(par):
            pltpu.async_copy(rowA.at[par], accsh.at[dsts.at[par]],
                             sems.at[par], add=True)
            pltpu.async_copy(denb.at[par], densh.at[dsts.at[par]],
                             sems.at[par], add=True)

        def drain_scatter(par):
            pltpu.make_async_copy(rowA.at[par], accsh.at[pl.ds(0, KCL)],
                                  sems.at[par]).wait()
            pltpu.make_async_copy(denb.at[par], densh.at[pl.ds(0, KCL)],
                                  sems.at[par]).wait()

        def zrow(r, carry):
            for t in range(TCH):
                rowA[0, r, pl.ds(t * 16, 16)] = zero16
            denb[0, r, pl.ds(0, 16)] = zero16
            return carry

        lax.fori_loop(0, KCL, zrow, 0)
        base = s * RPT
        for i in range(RPT // KCL):
            pltpu.sync_copy(rowA.at[0], accsh.at[pl.ds(base + i * KCL, KCL)])
            pltpu.sync_copy(denb.at[0], densh.at[pl.ds(base + i * KCL, KCL)])
        rem = RPT - (RPT // KCL) * KCL
        if rem:
            lastz = pl.ds(base + RPT - rem, rem)
            pltpu.sync_copy(rowA.at[0].at[pl.ds(0, rem)], accsh.at[lastz])
            pltpu.sync_copy(denb.at[0].at[pl.ds(0, rem)], densh.at[lastz])
        plsc.subcore_barrier()

        issue_idx(0, 0)
        drain_idx(0)
        issue_gather(0)
        issue_idx(1, 1)

        def p3(j, carry):
            b0 = lax.rem(j, 3)
            b1 = lax.rem(j + 1, 3)
            b2 = lax.rem(j + 2, 3)

            @pl.when(j >= 2)
            def _():
                drain_scatter(b1)

            @pl.when(j + 1 < NCHL)
            def _():
                drain_idx(b1)
                issue_gather(b1)

            drain_gather(b0)

            @pl.when(j + 2 < NCHL)
            def _():
                issue_idx(j + 2, b2)

            for k in range(KCL // 16):
                sl = pl.ds(k * 16, 16)
                dsts[b0, sl] = dstb[b0, sl]
            for k in range(KCL // 16):
                pv = pbuf[b0, pl.ds(k * 16, 16)]
                for kk in range(16):
                    r = k * 16 + kk
                    ps = pv[kk]
                    for t in range(TCH):
                        slt = pl.ds(t * 16, 16)
                        rowA[b0, r, slt] = rowA[b0, r, slt] * ps
                    denb[b0, r, pl.ds(0, 16)] = lane0 * ps
            issue_scatter(b0)
            return carry

        lax.fori_loop(0, NCHL, p3, 0)
        drain_scatter((NCHL - 2) % 3)
        drain_scatter((NCHL - 1) % 3)
        plsc.subcore_barrier()

        pltpu.sync_copy(accsh.at[pl.ds(base, RPT)],
                        acc_hbm.at[c].at[pl.ds(base, RPT)])
        pltpu.sync_copy(densh.at[pl.ds(base, RPT)],
                        den_hbm.at[c].at[pl.ds(base, RPT)])

    return sck


def _dense1_body(x_ref, w_ref, as_ref, ad_ref, ew_ref,
                 h_ref, asp_ref, adp_ref, ews_ref):
    h = jnp.dot(x_ref[...], w_ref[...], preferred_element_type=jnp.float32)
    h_ref[...] = h
    asp_ref[...] = jnp.sum(h * as_ref[...], axis=1, keepdims=True)
    adp_ref[...] = jnp.sum(h * ad_ref[...], axis=1, keepdims=True)

    @pl.when(pl.program_id(0) == 0)
    def _():
        ews_ref[...] = jnp.zeros_like(ews_ref)

    ews_ref[...] = ews_ref[...] + jnp.sum(ew_ref[...])


def _dense1(x, W, a_s, a_d, ew):
    dout = W.shape[1]
    return pl.pallas_call(
        _dense1_body,
        grid=(GRID,),
        in_specs=[
            pl.BlockSpec((BN, W.shape[0]), lambda i: (i, 0)),
            pl.BlockSpec(W.shape, lambda i: (0, 0)),
            pl.BlockSpec((1, dout), lambda i: (0, 0)),
            pl.BlockSpec((1, dout), lambda i: (0, 0)),
            pl.BlockSpec((E // GRID, 1), lambda i: (i, 0)),
        ],
        out_specs=[
            pl.BlockSpec((BN, dout), lambda i: (i, 0)),
            pl.BlockSpec((BN, 1), lambda i: (i, 0)),
            pl.BlockSpec((BN, 1), lambda i: (i, 0)),
            pl.BlockSpec((1, 1), lambda i: (0, 0)),
        ],
        out_shape=[
            jax.ShapeDtypeStruct((N, dout), jnp.float32),
            jax.ShapeDtypeStruct((N, 1), jnp.float32),
            jax.ShapeDtypeStruct((N, 1), jnp.float32),
            jax.ShapeDtypeStruct((1, 1), jnp.float32),
        ],
        compiler_params=pltpu.CompilerParams(
            dimension_semantics=("arbitrary",)),
    )(x, W, a_s, a_d, ew)


def _combine_xin(Dp, acc0_ref, acc1_ref, den0_ref, den1_ref, m_ref, hp_ref,
                 asp_ref, adp_ref, bp_ref, lm_ref):
    m0 = jnp.max(m_ref[...][0:1, :])
    m1 = jnp.max(m_ref[...][1:2, :])
    g = jnp.maximum(m0, m1)
    f0 = jnp.exp(m0 - g)
    f1 = jnp.exp(m1 - g)
    al = asp_ref[...] + adp_ref[...] + lm_ref[...]
    al = jnp.where(al >= 0.0, al, 0.2 * al)
    ploop = jnp.exp(al - g)
    num = acc0_ref[...] * f0 + acc1_ref[...] * f1 + hp_ref[...] * ploop
    den = (den0_ref[...][:, 0:1] * f0 + den1_ref[...][:, 0:1] * f1 + ploop)
    xin = num / den + bp_ref[...]
    return jnp.where(xin >= 0.0, xin, 0.01 * xin)


def _make_combine_matmul(Dp, Dn):
    def body(acc0_ref, acc1_ref, den0_ref, den1_ref, m_ref, hp_ref, asp_ref,
             adp_ref, bp_ref, lm_ref, w_ref, as_ref, ad_ref,
             h_ref, aspo_ref, adpo_ref):
        xin = _combine_xin(Dp, acc0_ref, acc1_ref, den0_ref, den1_ref, m_ref,
                           hp_ref, asp_ref, adp_ref, bp_ref, lm_ref)
        h = jnp.dot(xin, w_ref[...], preferred_element_type=jnp.float32)
        h_ref[...] = h
        aspo_ref[...] = jnp.sum(h * as_ref[...], axis=1, keepdims=True)
        adpo_ref[...] = jnp.sum(h * ad_ref[...], axis=1, keepdims=True)

    def run(acc, den, m, hp, asp, adp, bp, lm, W, a_s, a_d):
        return pl.pallas_call(
            body,
            grid=(GRID,),
            in_specs=[
                pl.BlockSpec((BN, Dp), lambda i: (i, 0)),
                pl.BlockSpec((BN, Dp), lambda i: (i, 0)),
                pl.BlockSpec((BN, 16), lambda i: (i, 0)),
                pl.BlockSpec((BN, 16), lambda i: (i, 0)),
                pl.BlockSpec((2, 16), lambda i: (0, 0)),
                pl.BlockSpec((BN, Dp), lambda i: (i, 0)),
                pl.BlockSpec((BN, 1), lambda i: (i, 0)),
                pl.BlockSpec((BN, 1), lambda i: (i, 0)),
                pl.BlockSpec((1, Dp), lambda i: (0, 0)),
                pl.BlockSpec((1, 1), lambda i: (0, 0)),
                pl.BlockSpec((Dp, Dn), lambda i: (0, 0)),
                pl.BlockSpec((1, Dn), lambda i: (0, 0)),
                pl.BlockSpec((1, Dn), lambda i: (0, 0)),
            ],
            out_specs=[
                pl.BlockSpec((BN, Dn), lambda i: (i, 0)),
                pl.BlockSpec((BN, 1), lambda i: (i, 0)),
                pl.BlockSpec((BN, 1), lambda i: (i, 0)),
            ],
            out_shape=[
                jax.ShapeDtypeStruct((N, Dn), jnp.float32),
                jax.ShapeDtypeStruct((N, 1), jnp.float32),
                jax.ShapeDtypeStruct((N, 1), jnp.float32),
            ],
        )(acc[0], acc[1], den[0], den[1], m, hp, asp, adp, bp, lm,
          W, a_s, a_d)

    return run


def _make_combine_out(Dp):
    def body(acc0_ref, acc1_ref, den0_ref, den1_ref, m_ref, hp_ref, asp_ref,
             adp_ref, bp_ref, lm_ref, o_ref):
        o_ref[...] = _combine_xin(Dp, acc0_ref, acc1_ref, den0_ref, den1_ref,
                                  m_ref, hp_ref, asp_ref, adp_ref, bp_ref,
                                  lm_ref)

    def run(acc, den, m, hp, asp, adp, bp, lm):
        return pl.pallas_call(
            body,
            grid=(GRID,),
            in_specs=[
                pl.BlockSpec((BN, Dp), lambda i: (i, 0)),
                pl.BlockSpec((BN, Dp), lambda i: (i, 0)),
                pl.BlockSpec((BN, 16), lambda i: (i, 0)),
                pl.BlockSpec((BN, 16), lambda i: (i, 0)),
                pl.BlockSpec((2, 16), lambda i: (0, 0)),
                pl.BlockSpec((BN, Dp), lambda i: (i, 0)),
                pl.BlockSpec((BN, 1), lambda i: (i, 0)),
                pl.BlockSpec((BN, 1), lambda i: (i, 0)),
                pl.BlockSpec((1, Dp), lambda i: (0, 0)),
                pl.BlockSpec((1, 1), lambda i: (0, 0)),
            ],
            out_specs=pl.BlockSpec((BN, Dp), lambda i: (i, 0)),
            out_shape=jax.ShapeDtypeStruct((N, Dp), jnp.float32),
        )(acc[0], acc[1], den[0], den[1], m, hp, asp, adp, bp, lm)

    return run


_sc_alpha = _make_sc_alpha()
_KC1, _KC2, _KC3 = 80, 80, 80
_sc_scat128 = _make_sc_scatter(128, _KC1)
_sc_scat64 = _make_sc_scatter(64, _KC2)
_sc_scat32 = _make_sc_scatter(32, _KC3)
_comb12 = _make_combine_matmul(128, 64)
_comb23 = _make_combine_matmul(64, 32)
_comb3o = _make_combine_out(32)


def kernel(x, adj, edge_weight, W1, as1, ad1, We1, ae1, b1,
           W2, as2, ad2, We2, ae2, b2, W3, as3, ad3, We3, ae3, b3):
    srcf = adj[0].reshape(NTILES, EW)
    dstf = adj[1].reshape(NTILES, EW)
    ewf = edge_weight.reshape(NTILES, EW)

    c1 = jnp.sum(We1[0] * ae1)
    c2 = jnp.sum(We2[0] * ae2)
    c3 = jnp.sum(We3[0] * ae3)

    def gat_sc(h, asp, adp, cval, scat, kcl):
        p, m = _sc_alpha(asp.reshape(N), adp.reshape(N), srcf, dstf, ewf,
                         jnp.full((16,), cval, jnp.float32))
        acc, den = scat(h, adj[0].reshape(NTILES, EW // kcl, kcl),
                        adj[1].reshape(NTILES, EW // kcl, kcl),
                        p.reshape(NTILES, EW // kcl, kcl))
        return acc, den, m

    h1, asp1, adp1, ews = _dense1(x, W1, as1.reshape(1, -1),
                                  ad1.reshape(1, -1), edge_weight)
    acc1, den1, m1 = gat_sc(h1, asp1, adp1, c1, _sc_scat128, _KC1)
    lm1 = (ews * (1.0 / E) * c1).reshape(1, 1)
    h2, asp2, adp2 = _comb12(acc1, den1, m1, h1, asp1, adp1,
                             b1.reshape(1, -1), lm1, W2,
                             as2.reshape(1, -1), ad2.reshape(1, -1))
    acc2, den2, m2 = gat_sc(h2, asp2, adp2, c2, _sc_scat64, _KC2)
    lm2 = (ews * (1.0 / E) * c2).reshape(1, 1)
    h3, asp3, adp3 = _comb23(acc2, den2, m2, h2, asp2, adp2,
                             b2.reshape(1, -1), lm2, W3,
                             as3.reshape(1, -1), ad3.reshape(1, -1))
    acc3, den3, m3 = gat_sc(h3, asp3, adp3, c3, _sc_scat32, _KC3)
    lm3 = (ews * (1.0 / E) * c3).reshape(1, 1)
    return _comb3o(acc3, den3, m3, h3, asp3, adp3, b3.reshape(1, -1), lm3)


# speedup vs baseline: 52.5157x; 1.2155x over previous
import functools

import jax
import jax.numpy as jnp
from jax import lax
from jax.experimental import pallas as pl
from jax.experimental.pallas import tpu as pltpu
from jax.experimental.pallas import tpu_sc as plsc

N = 10000
E = 320000
NTILES = 32
EW = E // NTILES
KC = 80
NCH = EW // KC
RPT = N // 16
BN = 1000
GRID = N // BN


_SC_PARAMS = pltpu.CompilerParams(use_tc_tiling_on_sc=False,
                                  needs_layout_passes=False)
_MESH = plsc.VectorSubcoreMesh(core_axis_name="c", subcore_axis_name="s")


def _make_sc_alpha():
    out_type = [
        jax.ShapeDtypeStruct((NTILES, EW), jnp.float32),
        jax.ShapeDtypeStruct((2, 16), jnp.float32),
    ]
    scratch = [
        pltpu.VMEM((N,), jnp.float32),
        pltpu.VMEM((N,), jnp.float32),
        pltpu.VMEM((EW,), jnp.int32),
        pltpu.VMEM((EW,), jnp.int32),
        pltpu.VMEM((EW,), jnp.float32),
        pltpu.VMEM((EW,), jnp.float32),
        pltpu.VMEM((16,), jnp.float32),
        pltpu.VMEM((16, 16), jnp.float32),
        pltpu.VMEM((16,), jnp.float32),
        pltpu.VMEM_SHARED((16, 16), jnp.float32),
    ]

    @functools.partial(pl.kernel, out_type=out_type, mesh=_MESH,
                       scratch_types=scratch, compiler_params=_SC_PARAMS)
    def sck(as_hbm, ad_hbm, src_hbm, dst_hbm, ew_hbm, c_hbm,
            p_hbm, m_hbm,
            asv, adv, srcf, dstf, ewf, av, mxv, mx2v, cv, mxsh):
        c = lax.axis_index("c")
        s = lax.axis_index("s")
        wid = s * 2 + c

        pltpu.sync_copy(as_hbm, asv)
        pltpu.sync_copy(ad_hbm, adv)
        pltpu.sync_copy(src_hbm.at[wid], srcf)
        pltpu.sync_copy(dst_hbm.at[wid], dstf)
        pltpu.sync_copy(ew_hbm.at[wid], ewf)
        pltpu.sync_copy(c_hbm, cv)
        cev = cv[...]

        def p1(j, mx):
            sl = pl.ds(j * 16, 16)
            asg = plsc.load_gather(asv, [srcf[sl]])
            adg = plsc.load_gather(adv, [dstf[sl]])
            a = asg + adg + cev * ewf[sl]
            a = jnp.where(a >= 0.0, a, 0.2 * a)
            av[sl] = a
            return jnp.maximum(mx, a)

        mx = lax.fori_loop(0, EW // 16, p1,
                           jnp.full((16,), -jnp.inf, jnp.float32))
        mxv[...] = mx
        pltpu.sync_copy(mxv, mxsh.at[s])
        plsc.subcore_barrier()
        pltpu.sync_copy(mxsh, mx2v)

        def pmax(k, mm):
            return jnp.maximum(mm, mx2v[k])

        mm = lax.fori_loop(0, 16, pmax,
                           jnp.full((16,), -jnp.inf, jnp.float32))
        msc = jnp.max(mm)

        @pl.when(s == 0)
        def _():
            mxv[...] = jnp.zeros((16,), jnp.float32) + msc
            pltpu.sync_copy(mxv, m_hbm.at[c])

        def p2(j, carry):
            sl = pl.ds(j * 16, 16)
            av[sl] = jnp.exp(av[sl] - msc)
            return carry

        lax.fori_loop(0, EW // 16, p2, 0)
        pltpu.sync_copy(av, p_hbm.at[wid])

    return sck


def _make_sc_gat(D, KCL):
    TCH = D // 16
    NCHL = EW // KCL
    out_type = [
        jax.ShapeDtypeStruct((2, N, D), jnp.float32),
        jax.ShapeDtypeStruct((2, N, 16), jnp.float32),
        jax.ShapeDtypeStruct((2, 16), jnp.float32),
    ]
    scratch = [
        pltpu.VMEM((N,), jnp.float32),
        pltpu.VMEM((N,), jnp.float32),
        pltpu.VMEM((EW,), jnp.int32),
        pltpu.VMEM((EW,), jnp.int32),
        pltpu.VMEM((EW,), jnp.float32),
        pltpu.VMEM((16,), jnp.float32),
        pltpu.VMEM((16, 16), jnp.float32),
        pltpu.VMEM((16,), jnp.float32),
        pltpu.VMEM((3, KCL), jnp.int32),
        pltpu.VMEM((3, KCL, D), jnp.float32),
        pltpu.VMEM((3, KCL, 16), jnp.float32),
        pltpu.VMEM_SHARED((N, D), jnp.float32),
        pltpu.VMEM_SHARED((N, 16), jnp.float32),
        pltpu.VMEM_SHARED((16, 16), jnp.float32),
        pltpu.SemaphoreType.DMA((3,)),
        pltpu.SemaphoreType.DMA((3,)),
    ]

    @functools.partial(pl.kernel, out_type=out_type, mesh=_MESH,
                       scratch_types=scratch, compiler_params=_SC_PARAMS)
    def sck(h_hbm, as_hbm, ad_hbm, src_hbm, dst_hbm, ew_hbm, c_hbm,
            acc_hbm, den_hbm, m_hbm,
            asv, adv, srcf, dstf, ewf, mxv, mx2v, cv, dsts, rowA, denb,
            accsh, densh, mxsh, semg, sems):
        c = lax.axis_index("c")
        s = lax.axis_index("s")
        wid = s * 2 + c
        zero16 = jnp.zeros((16,), jnp.float32)
        lane0 = (lax.iota(jnp.int32, 16) == 0).astype(jnp.float32)

        pltpu.sync_copy(as_hbm, asv)
        pltpu.sync_copy(ad_hbm, adv)
        pltpu.sync_copy(src_hbm.at[wid], srcf)
        pltpu.sync_copy(dst_hbm.at[wid], dstf)
        pltpu.sync_copy(ew_hbm.at[wid], ewf)
        pltpu.sync_copy(c_hbm, cv)
        cev = cv[...]

        def p1(j, mx):
            sl = pl.ds(j * 16, 16)
            asg = plsc.load_gather(asv, [srcf[sl]])
            adg = plsc.load_gather(adv, [dstf[sl]])
            a = asg + adg + cev * ewf[sl]
            a = jnp.where(a >= 0.0, a, 0.2 * a)
            ewf[sl] = a
            return jnp.maximum(mx, a)

        mx = lax.fori_loop(0, EW // 16, p1,
                           jnp.full((16,), -jnp.inf, jnp.float32))
        mxv[...] = mx
        pltpu.sync_copy(mxv, mxsh.at[s])
        plsc.subcore_barrier()
        pltpu.sync_copy(mxsh, mx2v)

        def pmax(k, mm):
            return jnp.maximum(mm, mx2v[k])

        mm = lax.fori_loop(0, 16, pmax,
                           jnp.full((16,), -jnp.inf, jnp.float32))
        msc = jnp.max(mm)

        @pl.when(s == 0)
        def _():
            mxv[...] = jnp.zeros((16,), jnp.float32) + msc
            pltpu.sync_copy(mxv, m_hbm.at[c])

        def issue_gather(j, par):
            pltpu.async_copy(h_hbm.at[srcf.at[pl.ds(j * KCL, KCL)]],
                             rowA.at[par], semg.at[par])

        def drain_gather(par):
            pltpu.make_async_copy(h_hbm.at[pl.ds(0, KCL)], rowA.at[par],
                                  semg.at[par]).wait()

        def issue_scatter(par):
            pltpu.async_copy(rowA.at[par], accsh.at[dsts.at[par]],
                             sems.at[par], add=True)
            pltpu.async_copy(denb.at[par], densh.at[dsts.at[par]],
                             sems.at[par], add=True)

        def drain_scatter(par):
            pltpu.make_async_copy(rowA.at[par], accsh.at[pl.ds(0, KCL)],
                                  sems.at[par]).wait()
            pltpu.make_async_copy(denb.at[par], densh.at[pl.ds(0, KCL)],
                                  sems.at[par]).wait()

        def zrow(r, carry):
            for t in range(TCH):
                rowA[0, r, pl.ds(t * 16, 16)] = zero16
            denb[0, r, pl.ds(0, 16)] = zero16
            return carry

        lax.fori_loop(0, KCL, zrow, 0)
        base = s * RPT
        for i in range(RPT // KCL):
            pltpu.sync_copy(rowA.at[0], accsh.at[pl.ds(base + i * KCL, KCL)])
            pltpu.sync_copy(denb.at[0], densh.at[pl.ds(base + i * KCL, KCL)])
        rem = RPT - (RPT // KCL) * KCL
        if rem:
            lastz = pl.ds(base + RPT - rem, rem)
            pltpu.sync_copy(rowA.at[0].at[pl.ds(0, rem)], accsh.at[lastz])
            pltpu.sync_copy(denb.at[0].at[pl.ds(0, rem)], densh.at[lastz])
        plsc.subcore_barrier()

        issue_gather(0, 0)

        def p3(j, carry):
            b0 = lax.rem(j, 3)
            b1 = lax.rem(j + 1, 3)

            @pl.when(j >= 2)
            def _():
                drain_scatter(b1)

            @pl.when(j + 1 < NCHL)
            def _():
                issue_gather(j + 1, b1)

            drain_gather(b0)
            for k in range(KCL // 16):
                sl = pl.ds(k * 16, 16)
                dsts[b0, sl] = dstf[pl.ds(j * KCL + k * 16, 16)]
            for k in range(KCL // 16):
                pv = jnp.exp(ewf[pl.ds(j * KCL + k * 16, 16)] - msc)
                for kk in range(16):
                    r = k * 16 + kk
                    ps = pv[kk]
                    for t in range(TCH):
                        slt = pl.ds(t * 16, 16)
                        rowA[b0, r, slt] = rowA[b0, r, slt] * ps
                    denb[b0, r, pl.ds(0, 16)] = lane0 * ps
            issue_scatter(b0)
            return carry

        lax.fori_loop(0, NCHL, p3, 0)
        drain_scatter((NCHL - 2) % 3)
        drain_scatter((NCHL - 1) % 3)
        plsc.subcore_barrier()

        pltpu.sync_copy(accsh.at[pl.ds(base, RPT)],
                        acc_hbm.at[c].at[pl.ds(base, RPT)])
        pltpu.sync_copy(densh.at[pl.ds(base, RPT)],
                        den_hbm.at[c].at[pl.ds(base, RPT)])

    return sck


def _make_sc_scatter(D, KCL):
    TCH = D // 16
    NCHL = EW // KCL
    out_type = [
        jax.ShapeDtypeStruct((2, N, D), jnp.float32),
        jax.ShapeDtypeStruct((2, N, 16), jnp.float32),
    ]
    scratch = [
        pltpu.VMEM((3, KCL), jnp.int32),
        pltpu.VMEM((3, KCL), jnp.int32),
        pltpu.VMEM((3, KCL), jnp.float32),
        pltpu.VMEM((3, KCL), jnp.int32),
        pltpu.VMEM((3, KCL, D), jnp.float32),
        pltpu.VMEM((3, KCL, 16), jnp.float32),
        pltpu.VMEM_SHARED((N, D), jnp.float32),
        pltpu.VMEM_SHARED((N, 16), jnp.float32),
        pltpu.SemaphoreType.DMA((3,)),
        pltpu.SemaphoreType.DMA((3,)),
        pltpu.SemaphoreType.DMA((3,)),
    ]

    @functools.partial(pl.kernel, out_type=out_type, mesh=_MESH,
                       scratch_types=scratch, compiler_params=_SC_PARAMS)
    def sck(h_hbm, src_hbm, dst_hbm, p_hbm,
            acc_hbm, den_hbm,
            srcb, dstb, pbuf, dsts, rowA, denb, accsh, densh,
            semi, semg, sems):
        c = lax.axis_index("c")
        s = lax.axis_index("s")
        wid = s * 2 + c
        zero16 = jnp.zeros((16,), jnp.float32)
        lane0 = (lax.iota(jnp.int32, 16) == 0).astype(jnp.float32)

        def issue_idx(j, par):
            pltpu.async_copy(src_hbm.at[wid, j], srcb.at[par], semi.at[par])
            pltpu.async_copy(dst_hbm.at[wid, j], dstb.at[par], semi.at[par])
            pltpu.async_copy(p_hbm.at[wid, j], pbuf.at[par], semi.at[par])

        def drain_idx(par):
            pltpu.make_async_copy(src_hbm.at[wid, 0], srcb.at[par],
                                  semi.at[par]).wait()
            pltpu.make_async_copy(dst_hbm.at[wid, 0], dstb.at[par],
                                  semi.at[par]).wait()
            pltpu.make_async_copy(p_hbm.at[wid, 0], pbuf.at[par],
                                  semi.at[par]).wait()

        def issue_gather(par):
            pltpu.async_copy(h_hbm.at[srcb.at[par]], rowA.at[par],
                             semg.at[par])

        def drain_gather(par):
            pltpu.make_async_copy(h_hbm.at[pl.ds(0, KCL)], rowA.at[par],
                                  semg.at[par]).wait()

        def issue_scatter(par):
            pltpu.async_copy(rowA.at[par], accsh.at[dsts.at[par]],
                             sems.at[par], add=True)
            pltpu.async_copy(denb.at[par], densh.at[dsts.at[par]],
                             sems.at[par], add=True)

        def drain_scatter(par):
            pltpu.make_async_copy(rowA.at[par], accsh.at[pl.ds(0, KCL)],
                                  sems.at[par]).wait()
            pltpu.make_async_copy(denb.at[par], densh.at[pl.ds(0, KCL)],
                                  sems.at[par]).wait()

        def zrow(r, carry):
            for t in range(TCH):
                rowA[0, r, pl.ds(t * 16, 16)] = zero16
            denb[0, r, pl.ds(0, 16)] = zero16
            return carry

        lax.fori_loop(0, KCL, zrow, 0)
        base = s * RPT
        for i in range(RPT // KCL):
            pltpu.sync_copy(rowA.at[0], accsh.at[pl.ds(base + i * KCL, KCL)])
            pltpu.sync_copy(denb.at[0], densh.at[pl.ds(base + i * KCL, KCL)])
        rem = RPT - (RPT // KCL) * KCL
        if rem:
            lastz = pl.ds(base + RPT - rem, rem)
            pltpu.sync_copy(rowA.at[0].at[pl.ds(0, rem)], accsh.at[lastz])
            pltpu.sync_copy(denb.at[0].at[pl.ds(0, rem)], densh.at[lastz])
        plsc.subcore_barrier()

        issue_idx(0, 0)
        drain_idx(0)
        issue_gather(0)
        issue_idx(1, 1)

        def p3(j, carry):
            b0 = lax.rem(j, 3)
            b1 = lax.rem(j + 1, 3)
            b2 = lax.rem(j + 2, 3)

            @pl.when(j >= 2)
            def _():
                drain_scatter(b1)

            @pl.when(j + 1 < NCHL)
            def _():
                drain_idx(b1)
                issue_gather(b1)

            drain_gather(b0)

            @pl.when(j + 2 < NCHL)
            def _():
                issue_idx(j + 2, b2)

            for k in range(KCL // 16):
                sl = pl.ds(k * 16, 16)
                dsts[b0, sl] = dstb[b0, sl]
            for k in range(KCL // 16):
                pv = pbuf[b0, pl.ds(k * 16, 16)]
                for kk in range(16):
                    r = k * 16 + kk
                    ps = pv[kk]
                    for t in range(TCH):
                        slt = pl.ds(t * 16, 16)
                        rowA[b0, r, slt] = rowA[b0, r, slt] * ps
                    denb[b0, r, pl.ds(0, 16)] = lane0 * ps
            issue_scatter(b0)
            return carry

        lax.fori_loop(0, NCHL, p3, 0)
        drain_scatter((NCHL - 2) % 3)
        drain_scatter((NCHL - 1) % 3)
        plsc.subcore_barrier()

        pltpu.sync_copy(accsh.at[pl.ds(base, RPT)],
                        acc_hbm.at[c].at[pl.ds(base, RPT)])
        pltpu.sync_copy(densh.at[pl.ds(base, RPT)],
                        den_hbm.at[c].at[pl.ds(base, RPT)])

    return sck


def _dense1_body(x_ref, w_ref, as_ref, ad_ref, ew_ref,
                 h_ref, asp_ref, adp_ref, ews_ref):
    h = jnp.dot(x_ref[...], w_ref[...], preferred_element_type=jnp.float32)
    h_ref[...] = h
    asp_ref[...] = jnp.sum(h * as_ref[...], axis=1, keepdims=True)
    adp_ref[...] = jnp.sum(h * ad_ref[...], axis=1, keepdims=True)

    @pl.when(pl.program_id(0) == 0)
    def _():
        ews_ref[...] = jnp.zeros_like(ews_ref) + jnp.sum(ew_ref[...])


def _dense1(x, W, a_s, a_d, ew):
    dout = W.shape[1]
    return pl.pallas_call(
        _dense1_body,
        grid=(GRID,),
        in_specs=[
            pl.BlockSpec((BN, W.shape[0]), lambda i: (i, 0)),
            pl.BlockSpec(W.shape, lambda i: (0, 0)),
            pl.BlockSpec((1, dout), lambda i: (0, 0)),
            pl.BlockSpec((1, dout), lambda i: (0, 0)),
            pl.BlockSpec((E // 128, 128), lambda i: (0, 0)),
        ],
        out_specs=[
            pl.BlockSpec((BN, dout), lambda i: (i, 0)),
            pl.BlockSpec((BN, 1), lambda i: (i, 0)),
            pl.BlockSpec((BN, 1), lambda i: (i, 0)),
            pl.BlockSpec((1, 1), lambda i: (0, 0)),
        ],
        out_shape=[
            jax.ShapeDtypeStruct((N, dout), jnp.float32),
            jax.ShapeDtypeStruct((N, 1), jnp.float32),
            jax.ShapeDtypeStruct((N, 1), jnp.float32),
            jax.ShapeDtypeStruct((1, 1), jnp.float32),
        ],
        compiler_params=pltpu.CompilerParams(
            dimension_semantics=("arbitrary",)),
    )(x, W, a_s, a_d, ew.reshape(E // 128, 128))


def _combine_xin(Dp, acc0_ref, acc1_ref, den0_ref, den1_ref, m_ref, hp_ref,
                 asp_ref, adp_ref, bp_ref, lm_ref):
    m0 = jnp.max(m_ref[...][0:1, :])
    m1 = jnp.max(m_ref[...][1:2, :])
    g = jnp.maximum(m0, m1)
    f0 = jnp.exp(m0 - g)
    f1 = jnp.exp(m1 - g)
    al = asp_ref[...] + adp_ref[...] + lm_ref[...]
    al = jnp.where(al >= 0.0, al, 0.2 * al)
    ploop = jnp.exp(al - g)
    num = acc0_ref[...] * f0 + acc1_ref[...] * f1 + hp_ref[...] * ploop
    den = (den0_ref[...][:, 0:1] * f0 + den1_ref[...][:, 0:1] * f1 + ploop)
    xin = num / den + bp_ref[...]
    return jnp.where(xin >= 0.0, xin, 0.01 * xin)


def _make_combine_matmul(Dp, Dn):
    def body(acc0_ref, acc1_ref, den0_ref, den1_ref, m_ref, hp_ref, asp_ref,
             adp_ref, bp_ref, lm_ref, w_ref, as_ref, ad_ref,
             h_ref, aspo_ref, adpo_ref):
        xin = _combine_xin(Dp, acc0_ref, acc1_ref, den0_ref, den1_ref, m_ref,
                           hp_ref, asp_ref, adp_ref, bp_ref, lm_ref)
        h = jnp.dot(xin, w_ref[...], preferred_element_type=jnp.float32)
        h_ref[...] = h
        aspo_ref[...] = jnp.sum(h * as_ref[...], axis=1, keepdims=True)
        adpo_ref[...] = jnp.sum(h * ad_ref[...], axis=1, keepdims=True)

    def run(acc, den, m, hp, asp, adp, bp, lm, W, a_s, a_d):
        return pl.pallas_call(
            body,
            grid=(GRID,),
            in_specs=[
                pl.BlockSpec((BN, Dp), lambda i: (i, 0)),
                pl.BlockSpec((BN, Dp), lambda i: (i, 0)),
                pl.BlockSpec((BN, 16), lambda i: (i, 0)),
                pl.BlockSpec((BN, 16), lambda i: (i, 0)),
                pl.BlockSpec((2, 16), lambda i: (0, 0)),
                pl.BlockSpec((BN, Dp), lambda i: (i, 0)),
                pl.BlockSpec((BN, 1), lambda i: (i, 0)),
                pl.BlockSpec((BN, 1), lambda i: (i, 0)),
                pl.BlockSpec((1, Dp), lambda i: (0, 0)),
                pl.BlockSpec((1, 1), lambda i: (0, 0)),
                pl.BlockSpec((Dp, Dn), lambda i: (0, 0)),
                pl.BlockSpec((1, Dn), lambda i: (0, 0)),
                pl.BlockSpec((1, Dn), lambda i: (0, 0)),
            ],
            out_specs=[
                pl.BlockSpec((BN, Dn), lambda i: (i, 0)),
                pl.BlockSpec((BN, 1), lambda i: (i, 0)),
                pl.BlockSpec((BN, 1), lambda i: (i, 0)),
            ],
            out_shape=[
                jax.ShapeDtypeStruct((N, Dn), jnp.float32),
                jax.ShapeDtypeStruct((N, 1), jnp.float32),
                jax.ShapeDtypeStruct((N, 1), jnp.float32),
            ],
        )(acc[0], acc[1], den[0], den[1], m, hp, asp, adp, bp, lm,
          W, a_s, a_d)

    return run


def _make_combine_out(Dp):
    def body(acc0_ref, acc1_ref, den0_ref, den1_ref, m_ref, hp_ref, asp_ref,
             adp_ref, bp_ref, lm_ref, o_ref):
        o_ref[...] = _combine_xin(Dp, acc0_ref, acc1_ref, den0_ref, den1_ref,
                                  m_ref, hp_ref, asp_ref, adp_ref, bp_ref,
                                  lm_ref)

    def run(acc, den, m, hp, asp, adp, bp, lm):
        return pl.pallas_call(
            body,
            grid=(GRID,),
            in_specs=[
                pl.BlockSpec((BN, Dp), lambda i: (i, 0)),
                pl.BlockSpec((BN, Dp), lambda i: (i, 0)),
                pl.BlockSpec((BN, 16), lambda i: (i, 0)),
                pl.BlockSpec((BN, 16), lambda i: (i, 0)),
                pl.BlockSpec((2, 16), lambda i: (0, 0)),
                pl.BlockSpec((BN, Dp), lambda i: (i, 0)),
                pl.BlockSpec((BN, 1), lambda i: (i, 0)),
                pl.BlockSpec((BN, 1), lambda i: (i, 0)),
                pl.BlockSpec((1, Dp), lambda i: (0, 0)),
                pl.BlockSpec((1, 1), lambda i: (0, 0)),
            ],
            out_specs=pl.BlockSpec((BN, Dp), lambda i: (i, 0)),
            out_shape=jax.ShapeDtypeStruct((N, Dp), jnp.float32),
        )(acc[0], acc[1], den[0], den[1], m, hp, asp, adp, bp, lm)

    return run


_sc_alpha = _make_sc_alpha()
_KC1, _KC2, _KC3 = 80, 80, 80
_sc_scat128 = _make_sc_scatter(128, _KC1)
_sc_gat64 = _make_sc_gat(64, _KC2)
_sc_gat32 = _make_sc_gat(32, _KC3)
_comb12 = _make_combine_matmul(128, 64)
_comb23 = _make_combine_matmul(64, 32)
_comb3o = _make_combine_out(32)


def kernel(x, adj, edge_weight, W1, as1, ad1, We1, ae1, b1,
           W2, as2, ad2, We2, ae2, b2, W3, as3, ad3, We3, ae3, b3):
    srcf = adj[0].reshape(NTILES, EW)
    dstf = adj[1].reshape(NTILES, EW)
    ewf = edge_weight.reshape(NTILES, EW)

    c1 = jnp.sum(We1[0] * ae1)
    c2 = jnp.sum(We2[0] * ae2)
    c3 = jnp.sum(We3[0] * ae3)

    def gat_sc1(h, asp, adp, cval, scat, kcl):
        p, m = _sc_alpha(asp.reshape(N), adp.reshape(N), srcf, dstf, ewf,
                         jnp.full((16,), cval, jnp.float32))
        acc, den = scat(h, adj[0].reshape(NTILES, EW // kcl, kcl),
                        adj[1].reshape(NTILES, EW // kcl, kcl),
                        p.reshape(NTILES, EW // kcl, kcl))
        return acc, den, m

    def gat_sc(h, asp, adp, cval, fused):
        return fused(h, asp.reshape(N), adp.reshape(N), srcf, dstf, ewf,
                     jnp.full((16,), cval, jnp.float32))

    h1, asp1, adp1, ews = _dense1(x, W1, as1.reshape(1, -1),
                                  ad1.reshape(1, -1), edge_weight)
    acc1, den1, m1 = gat_sc1(h1, asp1, adp1, c1, _sc_scat128, _KC1)
    lm1 = (ews * (1.0 / E) * c1).reshape(1, 1)
    h2, asp2, adp2 = _comb12(acc1, den1, m1, h1, asp1, adp1,
                             b1.reshape(1, -1), lm1, W2,
                             as2.reshape(1, -1), ad2.reshape(1, -1))
    acc2, den2, m2 = gat_sc(h2, asp2, adp2, c2, _sc_gat64)
    lm2 = (ews * (1.0 / E) * c2).reshape(1, 1)
    h3, asp3, adp3 = _comb23(acc2, den2, m2, h2, asp2, adp2,
                             b2.reshape(1, -1), lm2, W3,
                             as3.reshape(1, -1), ad3.reshape(1, -1))
    acc3, den3, m3 = gat_sc(h3, asp3, adp3, c3, _sc_gat32)
    lm3 = (ews * (1.0 / E) * c3).reshape(1, 1)
    return _comb3o(acc3, den3, m3, h3, asp3, adp3, b3.reshape(1, -1), lm3)


# speedup vs baseline: 53.6249x; 1.0211x over previous
import functools

import jax
import jax.numpy as jnp
from jax import lax
from jax.experimental import pallas as pl
from jax.experimental.pallas import tpu as pltpu
from jax.experimental.pallas import tpu_sc as plsc

N = 10000
E = 320000
NTILES = 32
EW = E // NTILES
KC = 80
NCH = EW // KC
RPT = N // 16
BN = 1000
GRID = N // BN


_SC_PARAMS = pltpu.CompilerParams(use_tc_tiling_on_sc=False,
                                  needs_layout_passes=False)
_MESH = plsc.VectorSubcoreMesh(core_axis_name="c", subcore_axis_name="s")


def _make_sc_alpha():
    out_type = [
        jax.ShapeDtypeStruct((NTILES, EW), jnp.float32),
        jax.ShapeDtypeStruct((2, 16), jnp.float32),
    ]
    scratch = [
        pltpu.VMEM((N,), jnp.float32),
        pltpu.VMEM((N,), jnp.float32),
        pltpu.VMEM((EW,), jnp.int32),
        pltpu.VMEM((EW,), jnp.int32),
        pltpu.VMEM((EW,), jnp.float32),
        pltpu.VMEM((EW,), jnp.float32),
        pltpu.VMEM((16,), jnp.float32),
        pltpu.VMEM((16, 16), jnp.float32),
        pltpu.VMEM((1, 16), jnp.float32),
        pltpu.VMEM_SHARED((16, 16), jnp.float32),
    ]

    @functools.partial(pl.kernel, out_type=out_type, mesh=_MESH,
                       scratch_types=scratch, compiler_params=_SC_PARAMS)
    def sck(as_hbm, ad_hbm, adj_hbm, ew_hbm, c_hbm,
            p_hbm, m_hbm,
            asv, adv, srcf, dstf, ewf, av, mxv, mx2v, cv, mxsh):
        c = lax.axis_index("c")
        s = lax.axis_index("s")
        wid = s * 2 + c

        pltpu.sync_copy(as_hbm, asv)
        pltpu.sync_copy(ad_hbm, adv)
        pltpu.sync_copy(adj_hbm.at[0, wid], srcf)
        pltpu.sync_copy(adj_hbm.at[1, wid], dstf)
        pltpu.sync_copy(ew_hbm.at[wid], ewf)
        pltpu.sync_copy(c_hbm, cv)
        cev = cv[0]

        def p1(j, mx):
            sl = pl.ds(j * 16, 16)
            asg = plsc.load_gather(asv, [srcf[sl]])
            adg = plsc.load_gather(adv, [dstf[sl]])
            a = asg + adg + cev * ewf[sl]
            a = jnp.where(a >= 0.0, a, 0.2 * a)
            av[sl] = a
            return jnp.maximum(mx, a)

        mx = lax.fori_loop(0, EW // 16, p1,
                           jnp.full((16,), -jnp.inf, jnp.float32))
        mxv[...] = mx
        pltpu.sync_copy(mxv, mxsh.at[s])
        plsc.subcore_barrier()
        pltpu.sync_copy(mxsh, mx2v)

        def pmax(k, mm):
            return jnp.maximum(mm, mx2v[k])

        mm = lax.fori_loop(0, 16, pmax,
                           jnp.full((16,), -jnp.inf, jnp.float32))
        msc = jnp.max(mm)

        @pl.when(s == 0)
        def _():
            mxv[...] = jnp.zeros((16,), jnp.float32) + msc
            pltpu.sync_copy(mxv, m_hbm.at[c])

        def p2(j, carry):
            sl = pl.ds(j * 16, 16)
            av[sl] = jnp.exp(av[sl] - msc)
            return carry

        lax.fori_loop(0, EW // 16, p2, 0)
        pltpu.sync_copy(av, p_hbm.at[wid])

    return sck


def _make_sc_gat(D, KCL):
    TCH = D // 16
    NCHL = EW // KCL
    out_type = [
        jax.ShapeDtypeStruct((2, N, D), jnp.float32),
        jax.ShapeDtypeStruct((2, N, 16), jnp.float32),
        jax.ShapeDtypeStruct((2, 16), jnp.float32),
    ]
    scratch = [
        pltpu.VMEM((N,), jnp.float32),
        pltpu.VMEM((N,), jnp.float32),
        pltpu.VMEM((EW,), jnp.int32),
        pltpu.VMEM((EW,), jnp.int32),
        pltpu.VMEM((EW,), jnp.float32),
        pltpu.VMEM((16,), jnp.float32),
        pltpu.VMEM((16, 16), jnp.float32),
        pltpu.VMEM((1, 16), jnp.float32),
        pltpu.VMEM((3, KCL), jnp.int32),
        pltpu.VMEM((3, KCL, D), jnp.float32),
        pltpu.VMEM((3, KCL, 16), jnp.float32),
        pltpu.VMEM_SHARED((N, D), jnp.float32),
        pltpu.VMEM_SHARED((N, 16), jnp.float32),
        pltpu.VMEM_SHARED((16, 16), jnp.float32),
        pltpu.SemaphoreType.DMA((3,)),
        pltpu.SemaphoreType.DMA((3,)),
    ]

    @functools.partial(pl.kernel, out_type=out_type, mesh=_MESH,
                       scratch_types=scratch, compiler_params=_SC_PARAMS)
    def sck(h_hbm, as_hbm, ad_hbm, adj_hbm, ew_hbm, c_hbm,
            acc_hbm, den_hbm, m_hbm,
            asv, adv, srcf, dstf, ewf, mxv, mx2v, cv, dsts, rowA, denb,
            accsh, densh, mxsh, semg, sems):
        c = lax.axis_index("c")
        s = lax.axis_index("s")
        wid = s * 2 + c
        zero16 = jnp.zeros((16,), jnp.float32)
        lane0 = (lax.iota(jnp.int32, 16) == 0).astype(jnp.float32)

        pltpu.sync_copy(as_hbm, asv)
        pltpu.sync_copy(ad_hbm, adv)
        pltpu.sync_copy(adj_hbm.at[0, wid], srcf)
        pltpu.sync_copy(adj_hbm.at[1, wid], dstf)
        pltpu.sync_copy(ew_hbm.at[wid], ewf)
        pltpu.sync_copy(c_hbm, cv)
        cev = cv[0]

        def p1(j, mx):
            sl = pl.ds(j * 16, 16)
            asg = plsc.load_gather(asv, [srcf[sl]])
            adg = plsc.load_gather(adv, [dstf[sl]])
            a = asg + adg + cev * ewf[sl]
            a = jnp.where(a >= 0.0, a, 0.2 * a)
            ewf[sl] = a
            return jnp.maximum(mx, a)

        mx = lax.fori_loop(0, EW // 16, p1,
                           jnp.full((16,), -jnp.inf, jnp.float32))
        mxv[...] = mx
        pltpu.sync_copy(mxv, mxsh.at[s])
        plsc.subcore_barrier()
        pltpu.sync_copy(mxsh, mx2v)

        def pmax(k, mm):
            return jnp.maximum(mm, mx2v[k])

        mm = lax.fori_loop(0, 16, pmax,
                           jnp.full((16,), -jnp.inf, jnp.float32))
        msc = jnp.max(mm)

        @pl.when(s == 0)
        def _():
            mxv[...] = jnp.zeros((16,), jnp.float32) + msc
            pltpu.sync_copy(mxv, m_hbm.at[c])

        def issue_gather(j, par):
            pltpu.async_copy(h_hbm.at[srcf.at[pl.ds(j * KCL, KCL)]],
                             rowA.at[par], semg.at[par])

        def drain_gather(par):
            pltpu.make_async_copy(h_hbm.at[pl.ds(0, KCL)], rowA.at[par],
                                  semg.at[par]).wait()

        def issue_scatter(par):
            pltpu.async_copy(rowA.at[par], accsh.at[dsts.at[par]],
                             sems.at[par], add=True)
            pltpu.async_copy(denb.at[par], densh.at[dsts.at[par]],
                             sems.at[par], add=True)

        def drain_scatter(par):
            pltpu.make_async_copy(rowA.at[par], accsh.at[pl.ds(0, KCL)],
                                  sems.at[par]).wait()
            pltpu.make_async_copy(denb.at[par], densh.at[pl.ds(0, KCL)],
                                  sems.at[par]).wait()

        def zrow(r, carry):
            for t in range(TCH):
                rowA[0, r, pl.ds(t * 16, 16)] = zero16
            denb[0, r, pl.ds(0, 16)] = zero16
            return carry

        lax.fori_loop(0, KCL, zrow, 0)
        base = s * RPT
        for i in range(RPT // KCL):
            pltpu.sync_copy(rowA.at[0], accsh.at[pl.ds(base + i * KCL, KCL)])
            pltpu.sync_copy(denb.at[0], densh.at[pl.ds(base + i * KCL, KCL)])
        rem = RPT - (RPT // KCL) * KCL
        if rem:
            lastz = pl.ds(base + RPT - rem, rem)
            pltpu.sync_copy(rowA.at[0].at[pl.ds(0, rem)], accsh.at[lastz])
            pltpu.sync_copy(denb.at[0].at[pl.ds(0, rem)], densh.at[lastz])
        plsc.subcore_barrier()

        issue_gather(0, 0)

        def p3(j, carry):
            b0 = lax.rem(j, 3)
            b1 = lax.rem(j + 1, 3)

            @pl.when(j >= 2)
            def _():
                drain_scatter(b1)

            @pl.when(j + 1 < NCHL)
            def _():
                issue_gather(j + 1, b1)

            drain_gather(b0)
            for k in range(KCL // 16):
                sl = pl.ds(k * 16, 16)
                dsts[b0, sl] = dstf[pl.ds(j * KCL + k * 16, 16)]
            for k in range(KCL // 16):
                pv = jnp.exp(ewf[pl.ds(j * KCL + k * 16, 16)] - msc)
                for kk in range(16):
                    r = k * 16 + kk
                    ps = pv[kk]
                    for t in range(TCH):
                        slt = pl.ds(t * 16, 16)
                        rowA[b0, r, slt] = rowA[b0, r, slt] * ps
                    denb[b0, r, pl.ds(0, 16)] = lane0 * ps
            issue_scatter(b0)
            return carry

        lax.fori_loop(0, NCHL, p3, 0)
        drain_scatter((NCHL - 2) % 3)
        drain_scatter((NCHL - 1) % 3)
        plsc.subcore_barrier()

        pltpu.sync_copy(accsh.at[pl.ds(base, RPT)],
                        acc_hbm.at[c].at[pl.ds(base, RPT)])
        pltpu.sync_copy(densh.at[pl.ds(base, RPT)],
                        den_hbm.at[c].at[pl.ds(base, RPT)])

    return sck


def _make_sc_scatter(D, KCL):
    TCH = D // 16
    NCHL = EW // KCL
    out_type = [
        jax.ShapeDtypeStruct((2, N, D), jnp.float32),
        jax.ShapeDtypeStruct((2, N, 16), jnp.float32),
    ]
    scratch = [
        pltpu.VMEM((3, KCL), jnp.int32),
        pltpu.VMEM((3, KCL), jnp.int32),
        pltpu.VMEM((3, KCL), jnp.float32),
        pltpu.VMEM((3, KCL), jnp.int32),
        pltpu.VMEM((3, KCL, D), jnp.float32),
        pltpu.VMEM((3, KCL, 16), jnp.float32),
        pltpu.VMEM_SHARED((N, D), jnp.float32),
        pltpu.VMEM_SHARED((N, 16), jnp.float32),
        pltpu.SemaphoreType.DMA((3,)),
        pltpu.SemaphoreType.DMA((3,)),
        pltpu.SemaphoreType.DMA((3,)),
    ]

    @functools.partial(pl.kernel, out_type=out_type, mesh=_MESH,
                       scratch_types=scratch, compiler_params=_SC_PARAMS)
    def sck(h_hbm, adj_hbm, p_hbm,
            acc_hbm, den_hbm,
            srcb, dstb, pbuf, dsts, rowA, denb, accsh, densh,
            semi, semg, sems):
        c = lax.axis_index("c")
        s = lax.axis_index("s")
        wid = s * 2 + c
        zero16 = jnp.zeros((16,), jnp.float32)
        lane0 = (lax.iota(jnp.int32, 16) == 0).astype(jnp.float32)

        def issue_idx(j, par):
            pltpu.async_copy(adj_hbm.at[0, wid, pl.ds(j * KCL, KCL)],
                             srcb.at[par], semi.at[par])
            pltpu.async_copy(adj_hbm.at[1, wid, pl.ds(j * KCL, KCL)],
                             dstb.at[par], semi.at[par])
            pltpu.async_copy(p_hbm.at[wid, pl.ds(j * KCL, KCL)],
                             pbuf.at[par], semi.at[par])

        def drain_idx(par):
            pltpu.make_async_copy(adj_hbm.at[0, wid, pl.ds(0, KCL)],
                                  srcb.at[par], semi.at[par]).wait()
            pltpu.make_async_copy(adj_hbm.at[1, wid, pl.ds(0, KCL)],
                                  dstb.at[par], semi.at[par]).wait()
            pltpu.make_async_copy(p_hbm.at[wid, pl.ds(0, KCL)],
                                  pbuf.at[par], semi.at[par]).wait()

        def issue_gather(par):
            pltpu.async_copy(h_hbm.at[srcb.at[par]], rowA.at[par],
                             semg.at[par])

        def drain_gather(par):
            pltpu.make_async_copy(h_hbm.at[pl.ds(0, KCL)], rowA.at[par],
                                  semg.at[par]).wait()

        def issue_scatter(par):
            pltpu.async_copy(rowA.at[par], accsh.at[dsts.at[par]],
                             sems.at[par], add=True)
            pltpu.async_copy(denb.at[par], densh.at[dsts.at[par]],
                             sems.at[par], add=True)

        def drain_scatter(par):
            pltpu.make_async_copy(rowA.at[par], accsh.at[pl.ds(0, KCL)],
                                  sems.at[par]).wait()
            pltpu.make_async_copy(denb.at[par], densh.at[pl.ds(0, KCL)],
                                  sems.at[par]).wait()

        def zrow(r, carry):
            for t in range(TCH):
                rowA[0, r, pl.ds(t * 16, 16)] = zero16
            denb[0, r, pl.ds(0, 16)] = zero16
            return carry

        lax.fori_loop(0, KCL, zrow, 0)
        base = s * RPT
        for i in range(RPT // KCL):
            pltpu.sync_copy(rowA.at[0], accsh.at[pl.ds(base + i * KCL, KCL)])
            pltpu.sync_copy(denb.at[0], densh.at[pl.ds(base + i * KCL, KCL)])
        rem = RPT - (RPT // KCL) * KCL
        if rem:
            lastz = pl.ds(base + RPT - rem, rem)
            pltpu.sync_copy(rowA.at[0].at[pl.ds(0, rem)], accsh.at[lastz])
            pltpu.sync_copy(denb.at[0].at[pl.ds(0, rem)], densh.at[lastz])
        plsc.subcore_barrier()

        issue_idx(0, 0)
        drain_idx(0)
        issue_gather(0)
        issue_idx(1, 1)

        def p3(j, carry):
            b0 = lax.rem(j, 3)
            b1 = lax.rem(j + 1, 3)
            b2 = lax.rem(j + 2, 3)

            @pl.when(j >= 2)
            def _():
                drain_scatter(b1)

            @pl.when(j + 1 < NCHL)
            def _():
                drain_idx(b1)
                issue_gather(b1)

            drain_gather(b0)

            @pl.when(j + 2 < NCHL)
            def _():
                issue_idx(j + 2, b2)

            for k in range(KCL // 16):
                sl = pl.ds(k * 16, 16)
                dsts[b0, sl] = dstb[b0, sl]
            for k in range(KCL // 16):
                pv = pbuf[b0, pl.ds(k * 16, 16)]
                for kk in range(16):
                    r = k * 16 + kk
                    ps = pv[kk]
                    for t in range(TCH):
                        slt = pl.ds(t * 16, 16)
                        rowA[b0, r, slt] = rowA[b0, r, slt] * ps
                    denb[b0, r, pl.ds(0, 16)] = lane0 * ps
            issue_scatter(b0)
            return carry

        lax.fori_loop(0, NCHL, p3, 0)
        drain_scatter((NCHL - 2) % 3)
        drain_scatter((NCHL - 1) % 3)
        plsc.subcore_barrier()

        pltpu.sync_copy(accsh.at[pl.ds(base, RPT)],
                        acc_hbm.at[c].at[pl.ds(base, RPT)])
        pltpu.sync_copy(densh.at[pl.ds(base, RPT)],
                        den_hbm.at[c].at[pl.ds(base, RPT)])

    return sck


def _dense1_body(x_ref, w_ref, as_ref, ad_ref, ew_ref, we1_ref, ae1_ref,
                 we2_ref, ae2_ref, we3_ref, ae3_ref,
                 h_ref, asp_ref, adp_ref, c1_ref, c2_ref, c3_ref,
                 lm1_ref, lm2_ref, lm3_ref):
    h = jnp.dot(x_ref[...], w_ref[...], preferred_element_type=jnp.float32)
    h_ref[...] = h
    asp_ref[...] = jnp.sum(h * as_ref[...], axis=1, keepdims=True)
    adp_ref[...] = jnp.sum(h * ad_ref[...], axis=1, keepdims=True)

    @pl.when(pl.program_id(0) == 0)
    def _():
        ewm = jnp.sum(ew_ref[...]) * (1.0 / E)
        ones = jnp.zeros((1, 16), jnp.float32)
        c1 = jnp.sum(we1_ref[...] * ae1_ref[...])
        c2 = jnp.sum(we2_ref[...] * ae2_ref[...])
        c3 = jnp.sum(we3_ref[...] * ae3_ref[...])
        c1_ref[...] = ones + c1
        c2_ref[...] = ones + c2
        c3_ref[...] = ones + c3
        lm1_ref[...] = jnp.zeros((1, 1), jnp.float32) + ewm * c1
        lm2_ref[...] = jnp.zeros((1, 1), jnp.float32) + ewm * c2
        lm3_ref[...] = jnp.zeros((1, 1), jnp.float32) + ewm * c3


def _dense1(x, W, a_s, a_d, ew, we1, ae1, we2, ae2, we3, ae3):
    dout = W.shape[1]
    full = lambda shape: pl.BlockSpec(shape, lambda i: (0, 0))
    return pl.pallas_call(
        _dense1_body,
        grid=(GRID,),
        in_specs=[
            pl.BlockSpec((BN, W.shape[0]), lambda i: (i, 0)),
            full(W.shape),
            full((1, dout)),
            full((1, dout)),
            full((E // 128, 128)),
            full((1, 128)), full((1, 128)),
            full((1, 64)), full((1, 64)),
            full((1, 32)), full((1, 32)),
        ],
        out_specs=[
            pl.BlockSpec((BN, dout), lambda i: (i, 0)),
            pl.BlockSpec((BN, 1), lambda i: (i, 0)),
            pl.BlockSpec((BN, 1), lambda i: (i, 0)),
            full((1, 16)), full((1, 16)), full((1, 16)),
            full((1, 1)), full((1, 1)), full((1, 1)),
        ],
        out_shape=[
            jax.ShapeDtypeStruct((N, dout), jnp.float32),
            jax.ShapeDtypeStruct((N, 1), jnp.float32),
            jax.ShapeDtypeStruct((N, 1), jnp.float32),
            jax.ShapeDtypeStruct((1, 16), jnp.float32),
            jax.ShapeDtypeStruct((1, 16), jnp.float32),
            jax.ShapeDtypeStruct((1, 16), jnp.float32),
            jax.ShapeDtypeStruct((1, 1), jnp.float32),
            jax.ShapeDtypeStruct((1, 1), jnp.float32),
            jax.ShapeDtypeStruct((1, 1), jnp.float32),
        ],
        compiler_params=pltpu.CompilerParams(
            dimension_semantics=("arbitrary",)),
    )(x, W, a_s, a_d, ew.reshape(E // 128, 128),
      we1, ae1.reshape(1, -1), we2, ae2.reshape(1, -1),
      we3, ae3.reshape(1, -1))


def _combine_xin(Dp, acc0_ref, acc1_ref, den0_ref, den1_ref, m_ref, hp_ref,
                 asp_ref, adp_ref, bp_ref, lm_ref):
    m0 = jnp.max(m_ref[...][0:1, :])
    m1 = jnp.max(m_ref[...][1:2, :])
    g = jnp.maximum(m0, m1)
    f0 = jnp.exp(m0 - g)
    f1 = jnp.exp(m1 - g)
    al = asp_ref[...] + adp_ref[...] + lm_ref[...]
    al = jnp.where(al >= 0.0, al, 0.2 * al)
    ploop = jnp.exp(al - g)
    num = acc0_ref[...] * f0 + acc1_ref[...] * f1 + hp_ref[...] * ploop
    den = (den0_ref[...][:, 0:1] * f0 + den1_ref[...][:, 0:1] * f1 + ploop)
    xin = num / den + bp_ref[...]
    return jnp.where(xin >= 0.0, xin, 0.01 * xin)


def _make_combine_matmul(Dp, Dn):
    def body(acc0_ref, acc1_ref, den0_ref, den1_ref, m_ref, hp_ref, asp_ref,
             adp_ref, bp_ref, lm_ref, w_ref, as_ref, ad_ref,
             h_ref, aspo_ref, adpo_ref):
        xin = _combine_xin(Dp, acc0_ref, acc1_ref, den0_ref, den1_ref, m_ref,
                           hp_ref, asp_ref, adp_ref, bp_ref, lm_ref)
        h = jnp.dot(xin, w_ref[...], preferred_element_type=jnp.float32)
        h_ref[...] = h
        aspo_ref[...] = jnp.sum(h * as_ref[...], axis=1, keepdims=True)
        adpo_ref[...] = jnp.sum(h * ad_ref[...], axis=1, keepdims=True)

    def run(acc, den, m, hp, asp, adp, bp, lm, W, a_s, a_d):
        return pl.pallas_call(
            body,
            grid=(GRID,),
            in_specs=[
                pl.BlockSpec((BN, Dp), lambda i: (i, 0)),
                pl.BlockSpec((BN, Dp), lambda i: (i, 0)),
                pl.BlockSpec((BN, 16), lambda i: (i, 0)),
                pl.BlockSpec((BN, 16), lambda i: (i, 0)),
                pl.BlockSpec((2, 16), lambda i: (0, 0)),
                pl.BlockSpec((BN, Dp), lambda i: (i, 0)),
                pl.BlockSpec((BN, 1), lambda i: (i, 0)),
                pl.BlockSpec((BN, 1), lambda i: (i, 0)),
                pl.BlockSpec((1, Dp), lambda i: (0, 0)),
                pl.BlockSpec((1, 1), lambda i: (0, 0)),
                pl.BlockSpec((Dp, Dn), lambda i: (0, 0)),
                pl.BlockSpec((1, Dn), lambda i: (0, 0)),
                pl.BlockSpec((1, Dn), lambda i: (0, 0)),
            ],
            out_specs=[
                pl.BlockSpec((BN, Dn), lambda i: (i, 0)),
                pl.BlockSpec((BN, 1), lambda i: (i, 0)),
                pl.BlockSpec((BN, 1), lambda i: (i, 0)),
            ],
            out_shape=[
                jax.ShapeDtypeStruct((N, Dn), jnp.float32),
                jax.ShapeDtypeStruct((N, 1), jnp.float32),
                jax.ShapeDtypeStruct((N, 1), jnp.float32),
            ],
        )(acc[0], acc[1], den[0], den[1], m, hp, asp, adp, bp, lm,
          W, a_s, a_d)

    return run


def _make_combine_out(Dp):
    def body(acc0_ref, acc1_ref, den0_ref, den1_ref, m_ref, hp_ref, asp_ref,
             adp_ref, bp_ref, lm_ref, o_ref):
        o_ref[...] = _combine_xin(Dp, acc0_ref, acc1_ref, den0_ref, den1_ref,
                                  m_ref, hp_ref, asp_ref, adp_ref, bp_ref,
                                  lm_ref)

    def run(acc, den, m, hp, asp, adp, bp, lm):
        return pl.pallas_call(
            body,
            grid=(GRID,),
            in_specs=[
                pl.BlockSpec((BN, Dp), lambda i: (i, 0)),
                pl.BlockSpec((BN, Dp), lambda i: (i, 0)),
                pl.BlockSpec((BN, 16), lambda i: (i, 0)),
                pl.BlockSpec((BN, 16), lambda i: (i, 0)),
                pl.BlockSpec((2, 16), lambda i: (0, 0)),
                pl.BlockSpec((BN, Dp), lambda i: (i, 0)),
                pl.BlockSpec((BN, 1), lambda i: (i, 0)),
                pl.BlockSpec((BN, 1), lambda i: (i, 0)),
                pl.BlockSpec((1, Dp), lambda i: (0, 0)),
                pl.BlockSpec((1, 1), lambda i: (0, 0)),
            ],
            out_specs=pl.BlockSpec((BN, Dp), lambda i: (i, 0)),
            out_shape=jax.ShapeDtypeStruct((N, Dp), jnp.float32),
        )(acc[0], acc[1], den[0], den[1], m, hp, asp, adp, bp, lm)

    return run


_sc_alpha = _make_sc_alpha()
_KC1, _KC2, _KC3 = 80, 80, 80
_sc_scat128 = _make_sc_scatter(128, _KC1)
_sc_gat64 = _make_sc_gat(64, _KC2)
_sc_gat32 = _make_sc_gat(32, _KC3)
_comb12 = _make_combine_matmul(128, 64)
_comb23 = _make_combine_matmul(64, 32)
_comb3o = _make_combine_out(32)


def kernel(x, adj, edge_weight, W1, as1, ad1, We1, ae1, b1,
           W2, as2, ad2, We2, ae2, b2, W3, as3, ad3, We3, ae3, b3):
    adjr = adj.reshape(2, NTILES, EW)
    ewr = edge_weight.reshape(NTILES, EW)

    h1, asp1, adp1, c1v, c2v, c3v, lm1, lm2, lm3 = _dense1(
        x, W1, as1.reshape(1, -1), ad1.reshape(1, -1), edge_weight,
        We1, ae1, We2, ae2, We3, ae3)

    p1, m1 = _sc_alpha(asp1.reshape(N), adp1.reshape(N), adjr, ewr, c1v)
    acc1, den1 = _sc_scat128(h1, adjr, p1)
    h2, asp2, adp2 = _comb12(acc1, den1, m1, h1, asp1, adp1,
                             b1.reshape(1, -1), lm1, W2,
                             as2.reshape(1, -1), ad2.reshape(1, -1))
    acc2, den2, m2 = _sc_gat64(h2, asp2.reshape(N), adp2.reshape(N),
                               adjr, ewr, c2v)
    h3, asp3, adp3 = _comb23(acc2, den2, m2, h2, asp2, adp2,
                             b2.reshape(1, -1), lm2, W3,
                             as3.reshape(1, -1), ad3.reshape(1, -1))
    acc3, den3, m3 = _sc_gat32(h3, asp3.reshape(N), adp3.reshape(N),
                               adjr, ewr, c3v)
    return _comb3o(acc3, den3, m3, h3, asp3, adp3, b3.reshape(1, -1), lm3)


# speedup vs baseline: 55.9773x; 1.0439x over previous
import functools

import jax
import jax.numpy as jnp
from jax import lax
from jax.experimental import pallas as pl
from jax.experimental.pallas import tpu as pltpu
from jax.experimental.pallas import tpu_sc as plsc

N = 10000
E = 320000
NTILES = 32
EW = E // NTILES
KC = 80
NCH = EW // KC
RPT = N // 16
BN = 1000
GRID = N // BN


_SC_PARAMS = pltpu.CompilerParams(use_tc_tiling_on_sc=False,
                                  needs_layout_passes=False)
_MESH = plsc.VectorSubcoreMesh(core_axis_name="c", subcore_axis_name="s")


def _make_sc_alpha():
    out_type = [
        jax.ShapeDtypeStruct((NTILES, EW), jnp.float32),
        jax.ShapeDtypeStruct((2, 16), jnp.float32),
    ]
    scratch = [
        pltpu.VMEM((2 * N,), jnp.float32),
        pltpu.VMEM((EW,), jnp.int32),
        pltpu.VMEM((EW,), jnp.int32),
        pltpu.VMEM((EW,), jnp.float32),
        pltpu.VMEM((EW,), jnp.float32),
        pltpu.VMEM((16,), jnp.float32),
        pltpu.VMEM((16, 16), jnp.float32),
        pltpu.VMEM((1, 16), jnp.float32),
        pltpu.VMEM_SHARED((16, 16), jnp.float32),
    ]

    @functools.partial(pl.kernel, out_type=out_type, mesh=_MESH,
                       scratch_types=scratch, compiler_params=_SC_PARAMS)
    def sck(aa_hbm, adj_hbm, ew_hbm, c_hbm,
            p_hbm, m_hbm,
            aav, srcf, dstf, ewf, av, mxv, mx2v, cv, mxsh):
        c = lax.axis_index("c")
        s = lax.axis_index("s")
        wid = s * 2 + c

        pltpu.sync_copy(aa_hbm, aav)
        pltpu.sync_copy(adj_hbm.at[0, wid], srcf)
        pltpu.sync_copy(adj_hbm.at[1, wid], dstf)
        pltpu.sync_copy(ew_hbm.at[wid], ewf)
        pltpu.sync_copy(c_hbm, cv)
        cev = cv[0]
        def p1(j, mx):
            sl = pl.ds(j * 16, 16)
            asg = plsc.load_gather(aav, [srcf[sl] * 2])
            adg = plsc.load_gather(aav, [dstf[sl] * 2 + 1])
            a = asg + adg + cev * ewf[sl]
            a = jnp.where(a >= 0.0, a, 0.2 * a)
            av[sl] = a
            return jnp.maximum(mx, a)

        mx = lax.fori_loop(0, EW // 16, p1,
                           jnp.full((16,), -jnp.inf, jnp.float32))
        mxv[...] = mx
        pltpu.sync_copy(mxv, mxsh.at[s])
        plsc.subcore_barrier()
        pltpu.sync_copy(mxsh, mx2v)

        def pmax(k, mm):
            return jnp.maximum(mm, mx2v[k])

        mm = lax.fori_loop(0, 16, pmax,
                           jnp.full((16,), -jnp.inf, jnp.float32))
        msc = jnp.max(mm)

        @pl.when(s == 0)
        def _():
            mxv[...] = jnp.zeros((16,), jnp.float32) + msc
            pltpu.sync_copy(mxv, m_hbm.at[c])

        def p2(j, carry):
            sl = pl.ds(j * 16, 16)
            av[sl] = jnp.exp(av[sl] - msc)
            return carry

        lax.fori_loop(0, EW // 16, p2, 0)
        pltpu.sync_copy(av, p_hbm.at[wid])

    return sck


def _make_sc_gat(D, KCL):
    TCH = D // 16
    NCHL = EW // KCL
    out_type = [
        jax.ShapeDtypeStruct((2, N, D), jnp.float32),
        jax.ShapeDtypeStruct((2, N, 16), jnp.float32),
        jax.ShapeDtypeStruct((2, 16), jnp.float32),
    ]
    scratch = [
        pltpu.VMEM((2 * N,), jnp.float32),
        pltpu.VMEM((EW,), jnp.int32),
        pltpu.VMEM((EW,), jnp.int32),
        pltpu.VMEM((EW,), jnp.float32),
        pltpu.VMEM((16,), jnp.float32),
        pltpu.VMEM((16, 16), jnp.float32),
        pltpu.VMEM((1, 16), jnp.float32),
        pltpu.VMEM((3, KCL), jnp.int32),
        pltpu.VMEM((3, KCL, D), jnp.float32),
        pltpu.VMEM((3, KCL, 16), jnp.float32),
        pltpu.VMEM_SHARED((N, D), jnp.float32),
        pltpu.VMEM_SHARED((N, 16), jnp.float32),
        pltpu.VMEM_SHARED((16, 16), jnp.float32),
        pltpu.SemaphoreType.DMA((3,)),
        pltpu.SemaphoreType.DMA((3,)),
    ]

    @functools.partial(pl.kernel, out_type=out_type, mesh=_MESH,
                       scratch_types=scratch, compiler_params=_SC_PARAMS)
    def sck(h_hbm, aa_hbm, adj_hbm, ew_hbm, c_hbm,
            acc_hbm, den_hbm, m_hbm,
            aav, srcf, dstf, ewf, mxv, mx2v, cv, dsts, rowA, denb,
            accsh, densh, mxsh, semg, sems):
        c = lax.axis_index("c")
        s = lax.axis_index("s")
        wid = s * 2 + c
        zero16 = jnp.zeros((16,), jnp.float32)
        lane0 = (lax.iota(jnp.int32, 16) == 0).astype(jnp.float32)
        pltpu.sync_copy(aa_hbm, aav)
        pltpu.sync_copy(adj_hbm.at[0, wid], srcf)
        pltpu.sync_copy(adj_hbm.at[1, wid], dstf)
        pltpu.sync_copy(ew_hbm.at[wid], ewf)
        pltpu.sync_copy(c_hbm, cv)
        cev = cv[0]

        def p1(j, mx):
            sl = pl.ds(j * 16, 16)
            asg = plsc.load_gather(aav, [srcf[sl] * 2])
            adg = plsc.load_gather(aav, [dstf[sl] * 2 + 1])
            a = asg + adg + cev * ewf[sl]
            a = jnp.where(a >= 0.0, a, 0.2 * a)
            ewf[sl] = a
            return jnp.maximum(mx, a)

        mx = lax.fori_loop(0, EW // 16, p1,
                           jnp.full((16,), -jnp.inf, jnp.float32))
        mxv[...] = mx
        pltpu.sync_copy(mxv, mxsh.at[s])
        plsc.subcore_barrier()
        pltpu.sync_copy(mxsh, mx2v)

        def pmax(k, mm):
            return jnp.maximum(mm, mx2v[k])

        mm = lax.fori_loop(0, 16, pmax,
                           jnp.full((16,), -jnp.inf, jnp.float32))
        msc = jnp.max(mm)

        @pl.when(s == 0)
        def _():
            mxv[...] = jnp.zeros((16,), jnp.float32) + msc
            pltpu.sync_copy(mxv, m_hbm.at[c])

        def issue_gather(j, par):
            pltpu.async_copy(h_hbm.at[srcf.at[pl.ds(j * KCL, KCL)]],
                             rowA.at[par], semg.at[par])

        def drain_gather(par):
            pltpu.make_async_copy(h_hbm.at[pl.ds(0, KCL)], rowA.at[par],
                                  semg.at[par]).wait()

        def issue_scatter(par):
            pltpu.async_copy(rowA.at[par], accsh.at[dsts.at[par]],
                             sems.at[par], add=True)
            pltpu.async_copy(denb.at[par], densh.at[dsts.at[par]],
                             sems.at[par], add=True)

        def drain_scatter(par):
            pltpu.make_async_copy(rowA.at[par], accsh.at[pl.ds(0, KCL)],
                                  sems.at[par]).wait()
            pltpu.make_async_copy(denb.at[par], densh.at[pl.ds(0, KCL)],
                                  sems.at[par]).wait()

        def zrow(r, carry):
            for t in range(TCH):
                rowA[0, r, pl.ds(t * 16, 16)] = zero16
            denb[0, r, pl.ds(0, 16)] = zero16
            return carry

        lax.fori_loop(0, KCL, zrow, 0)
        base = s * RPT
        for i in range(RPT // KCL):
            pltpu.sync_copy(rowA.at[0], accsh.at[pl.ds(base + i * KCL, KCL)])
            pltpu.sync_copy(denb.at[0], densh.at[pl.ds(base + i * KCL, KCL)])
        rem = RPT - (RPT // KCL) * KCL
        if rem:
            lastz = pl.ds(base + RPT - rem, rem)
            pltpu.sync_copy(rowA.at[0].at[pl.ds(0, rem)], accsh.at[lastz])
            pltpu.sync_copy(denb.at[0].at[pl.ds(0, rem)], densh.at[lastz])
        plsc.subcore_barrier()

        issue_gather(0, 0)

        def p3(j, carry):
            b0 = lax.rem(j, 3)
            b1 = lax.rem(j + 1, 3)

            @pl.when(j >= 2)
            def _():
                drain_scatter(b1)

            @pl.when(j + 1 < NCHL)
            def _():
                issue_gather(j + 1, b1)

            drain_gather(b0)
            for k in range(KCL // 16):
                sl = pl.ds(k * 16, 16)
                dsts[b0, sl] = dstf[pl.ds(j * KCL + k * 16, 16)]
            for k in range(KCL // 16):
                pv = jnp.exp(ewf[pl.ds(j * KCL + k * 16, 16)] - msc)
                for kk in range(16):
                    r = k * 16 + kk
                    ps = pv[kk]
                    for t in range(TCH):
                        slt = pl.ds(t * 16, 16)
                        rowA[b0, r, slt] = rowA[b0, r, slt] * ps
                    denb[b0, r, pl.ds(0, 16)] = lane0 * ps
            issue_scatter(b0)
            return carry

        lax.fori_loop(0, NCHL, p3, 0)
        drain_scatter((NCHL - 2) % 3)
        drain_scatter((NCHL - 1) % 3)
        plsc.subcore_barrier()

        pltpu.sync_copy(accsh.at[pl.ds(base, RPT)],
                        acc_hbm.at[c].at[pl.ds(base, RPT)])
        pltpu.sync_copy(densh.at[pl.ds(base, RPT)],
                        den_hbm.at[c].at[pl.ds(base, RPT)])

    return sck


def _make_sc_scatter(D, KCL):
    TCH = D // 16
    NCHL = EW // KCL
    out_type = [
        jax.ShapeDtypeStruct((2, N, D), jnp.float32),
        jax.ShapeDtypeStruct((2, N, 16), jnp.float32),
    ]
    scratch = [
        pltpu.VMEM((3, KCL), jnp.int32),
        pltpu.VMEM((3, KCL), jnp.int32),
        pltpu.VMEM((3, KCL), jnp.float32),
        pltpu.VMEM((3, KCL), jnp.int32),
        pltpu.VMEM((3, KCL, D), jnp.float32),
        pltpu.VMEM((3, KCL, 16), jnp.float32),
        pltpu.VMEM_SHARED((N, D), jnp.float32),
        pltpu.VMEM_SHARED((N, 16), jnp.float32),
        pltpu.SemaphoreType.DMA((3,)),
        pltpu.SemaphoreType.DMA((3,)),
        pltpu.SemaphoreType.DMA((3,)),
    ]

    @functools.partial(pl.kernel, out_type=out_type, mesh=_MESH,
                       scratch_types=scratch, compiler_params=_SC_PARAMS)
    def sck(h_hbm, adj_hbm, p_hbm,
            acc_hbm, den_hbm,
            srcb, dstb, pbuf, dsts, rowA, denb, accsh, densh,
            semi, semg, sems):
        c = lax.axis_index("c")
        s = lax.axis_index("s")
        wid = s * 2 + c
        zero16 = jnp.zeros((16,), jnp.float32)
        lane0 = (lax.iota(jnp.int32, 16) == 0).astype(jnp.float32)

        def issue_idx(j, par):
            pltpu.async_copy(adj_hbm.at[0, wid, pl.ds(j * KCL, KCL)],
                             srcb.at[par], semi.at[par])
            pltpu.async_copy(adj_hbm.at[1, wid, pl.ds(j * KCL, KCL)],
                             dstb.at[par], semi.at[par])
            pltpu.async_copy(p_hbm.at[wid, pl.ds(j * KCL, KCL)],
                             pbuf.at[par], semi.at[par])

        def drain_idx(par):
            pltpu.make_async_copy(adj_hbm.at[0, wid, pl.ds(0, KCL)],
                                  srcb.at[par], semi.at[par]).wait()
            pltpu.make_async_copy(adj_hbm.at[1, wid, pl.ds(0, KCL)],
                                  dstb.at[par], semi.at[par]).wait()
            pltpu.make_async_copy(p_hbm.at[wid, pl.ds(0, KCL)],
                                  pbuf.at[par], semi.at[par]).wait()

        def issue_gather(par):
            pltpu.async_copy(h_hbm.at[srcb.at[par]], rowA.at[par],
                             semg.at[par])

        def drain_gather(par):
            pltpu.make_async_copy(h_hbm.at[pl.ds(0, KCL)], rowA.at[par],
                                  semg.at[par]).wait()

        def issue_scatter(par):
            pltpu.async_copy(rowA.at[par], accsh.at[dsts.at[par]],
                             sems.at[par], add=True)
            pltpu.async_copy(denb.at[par], densh.at[dsts.at[par]],
                             sems.at[par], add=True)

        def drain_scatter(par):
            pltpu.make_async_copy(rowA.at[par], accsh.at[pl.ds(0, KCL)],
                                  sems.at[par]).wait()
            pltpu.make_async_copy(denb.at[par], densh.at[pl.ds(0, KCL)],
                                  sems.at[par]).wait()

        def zrow(r, carry):
            for t in range(TCH):
                rowA[0, r, pl.ds(t * 16, 16)] = zero16
            denb[0, r, pl.ds(0, 16)] = zero16
            return carry

        lax.fori_loop(0, KCL, zrow, 0)
        base = s * RPT
        for i in range(RPT // KCL):
            pltpu.sync_copy(rowA.at[0], accsh.at[pl.ds(base + i * KCL, KCL)])
            pltpu.sync_copy(denb.at[0], densh.at[pl.ds(base + i * KCL, KCL)])
        rem = RPT - (RPT // KCL) * KCL
        if rem:
            lastz = pl.ds(base + RPT - rem, rem)
            pltpu.sync_copy(rowA.at[0].at[pl.ds(0, rem)], accsh.at[lastz])
            pltpu.sync_copy(denb.at[0].at[pl.ds(0, rem)], densh.at[lastz])
        plsc.subcore_barrier()

        issue_idx(0, 0)
        drain_idx(0)
        issue_gather(0)
        issue_idx(1, 1)

        def p3(j, carry):
            b0 = lax.rem(j, 3)
            b1 = lax.rem(j + 1, 3)
            b2 = lax.rem(j + 2, 3)

            @pl.when(j >= 2)
            def _():
                drain_scatter(b1)

            @pl.when(j + 1 < NCHL)
            def _():
                drain_idx(b1)
                issue_gather(b1)

            drain_gather(b0)

            @pl.when(j + 2 < NCHL)
            def _():
                issue_idx(j + 2, b2)

            for k in range(KCL // 16):
                sl = pl.ds(k * 16, 16)
                dsts[b0, sl] = dstb[b0, sl]
            for k in range(KCL // 16):
                pv = pbuf[b0, pl.ds(k * 16, 16)]
                for kk in range(16):
                    r = k * 16 + kk
                    ps = pv[kk]
                    for t in range(TCH):
                        slt = pl.ds(t * 16, 16)
                        rowA[b0, r, slt] = rowA[b0, r, slt] * ps
                    denb[b0, r, pl.ds(0, 16)] = lane0 * ps
            issue_scatter(b0)
            return carry

        lax.fori_loop(0, NCHL, p3, 0)
        drain_scatter((NCHL - 2) % 3)
        drain_scatter((NCHL - 1) % 3)
        plsc.subcore_barrier()

        pltpu.sync_copy(accsh.at[pl.ds(base, RPT)],
                        acc_hbm.at[c].at[pl.ds(base, RPT)])
        pltpu.sync_copy(densh.at[pl.ds(base, RPT)],
                        den_hbm.at[c].at[pl.ds(base, RPT)])

    return sck


def _dense1_body(x_ref, w_ref, as_ref, ad_ref, ew_ref, we1_ref, ae1_ref,
                 we2_ref, ae2_ref, we3_ref, ae3_ref,
                 h_ref, aa_ref, c1_ref, c2_ref, c3_ref,
                 lm1_ref, lm2_ref, lm3_ref):
    h = jnp.dot(x_ref[...], w_ref[...], preferred_element_type=jnp.float32)
    h_ref[...] = h
    asp = jnp.sum(h * as_ref[...], axis=1, keepdims=True)
    adp = jnp.sum(h * ad_ref[...], axis=1, keepdims=True)
    aa_ref[...] = jnp.concatenate([asp, adp], axis=1)

    @pl.when(pl.program_id(0) == 0)
    def _():
        ewm = jnp.sum(ew_ref[...]) * (1.0 / E)
        ones = jnp.zeros((1, 16), jnp.float32)
        c1 = jnp.sum(we1_ref[...] * ae1_ref[...])
        c2 = jnp.sum(we2_ref[...] * ae2_ref[...])
        c3 = jnp.sum(we3_ref[...] * ae3_ref[...])
        c1_ref[...] = ones + c1
        c2_ref[...] = ones + c2
        c3_ref[...] = ones + c3
        lm1_ref[...] = jnp.zeros((1, 1), jnp.float32) + ewm * c1
        lm2_ref[...] = jnp.zeros((1, 1), jnp.float32) + ewm * c2
        lm3_ref[...] = jnp.zeros((1, 1), jnp.float32) + ewm * c3


def _dense1(x, W, a_s, a_d, ew, we1, ae1, we2, ae2, we3, ae3):
    dout = W.shape[1]
    full = lambda shape: pl.BlockSpec(shape, lambda i: (0, 0))
    return pl.pallas_call(
        _dense1_body,
        grid=(GRID,),
        in_specs=[
            pl.BlockSpec((BN, W.shape[0]), lambda i: (i, 0)),
            full(W.shape),
            full((1, dout)),
            full((1, dout)),
            full((E // 128, 128)),
            full((1, 128)), full((1, 128)),
            full((1, 64)), full((1, 64)),
            full((1, 32)), full((1, 32)),
        ],
        out_specs=[
            pl.BlockSpec((BN, dout), lambda i: (i, 0)),
            pl.BlockSpec((BN, 2), lambda i: (i, 0)),
            full((1, 16)), full((1, 16)), full((1, 16)),
            full((1, 1)), full((1, 1)), full((1, 1)),
        ],
        out_shape=[
            jax.ShapeDtypeStruct((N, dout), jnp.float32),
            jax.ShapeDtypeStruct((N, 2), jnp.float32),
            jax.ShapeDtypeStruct((1, 16), jnp.float32),
            jax.ShapeDtypeStruct((1, 16), jnp.float32),
            jax.ShapeDtypeStruct((1, 16), jnp.float32),
            jax.ShapeDtypeStruct((1, 1), jnp.float32),
            jax.ShapeDtypeStruct((1, 1), jnp.float32),
            jax.ShapeDtypeStruct((1, 1), jnp.float32),
        ],
        compiler_params=pltpu.CompilerParams(
            dimension_semantics=("arbitrary",)),
    )(x, W, a_s, a_d, ew.reshape(E // 128, 128),
      we1, ae1.reshape(1, -1), we2, ae2.reshape(1, -1),
      we3, ae3.reshape(1, -1))


def _combine_xin(Dp, acc0_ref, acc1_ref, den0_ref, den1_ref, m_ref, hp_ref,
                 aa_ref, bp_ref, lm_ref):
    m0 = jnp.max(m_ref[...][0:1, :])
    m1 = jnp.max(m_ref[...][1:2, :])
    g = jnp.maximum(m0, m1)
    f0 = jnp.exp(m0 - g)
    f1 = jnp.exp(m1 - g)
    al = aa_ref[...][:, 0:1] + aa_ref[...][:, 1:2] + lm_ref[...]
    al = jnp.where(al >= 0.0, al, 0.2 * al)
    ploop = jnp.exp(al - g)
    num = acc0_ref[...] * f0 + acc1_ref[...] * f1 + hp_ref[...] * ploop
    den = (den0_ref[...][:, 0:1] * f0 + den1_ref[...][:, 0:1] * f1 + ploop)
    xin = num / den + bp_ref[...]
    return jnp.where(xin >= 0.0, xin, 0.01 * xin)


def _make_combine_matmul(Dp, Dn):
    BC = 2000
    GC = N // BC

    def body(acc0_ref, acc1_ref, den0_ref, den1_ref, m_ref, hp_ref, aa_ref,
             bp_ref, lm_ref, w_ref, as_ref, ad_ref,
             h_ref, aao_ref):
        xin = _combine_xin(Dp, acc0_ref, acc1_ref, den0_ref, den1_ref, m_ref,
                           hp_ref, aa_ref, bp_ref, lm_ref)
        h = jnp.dot(xin, w_ref[...], preferred_element_type=jnp.float32)
        h_ref[...] = h
        asp = jnp.sum(h * as_ref[...], axis=1, keepdims=True)
        adp = jnp.sum(h * ad_ref[...], axis=1, keepdims=True)
        aao_ref[...] = jnp.concatenate([asp, adp], axis=1)

    def run(acc, den, m, hp, aa, bp, lm, W, a_s, a_d):
        return pl.pallas_call(
            body,
            grid=(GC,),
            in_specs=[
                pl.BlockSpec((BC, Dp), lambda i: (i, 0)),
                pl.BlockSpec((BC, Dp), lambda i: (i, 0)),
                pl.BlockSpec((BC, 16), lambda i: (i, 0)),
                pl.BlockSpec((BC, 16), lambda i: (i, 0)),
                pl.BlockSpec((2, 16), lambda i: (0, 0)),
                pl.BlockSpec((BC, Dp), lambda i: (i, 0)),
                pl.BlockSpec((BC, 2), lambda i: (i, 0)),
                pl.BlockSpec((1, Dp), lambda i: (0, 0)),
                pl.BlockSpec((1, 1), lambda i: (0, 0)),
                pl.BlockSpec((Dp, Dn), lambda i: (0, 0)),
                pl.BlockSpec((1, Dn), lambda i: (0, 0)),
                pl.BlockSpec((1, Dn), lambda i: (0, 0)),
            ],
            out_specs=[
                pl.BlockSpec((BC, Dn), lambda i: (i, 0)),
                pl.BlockSpec((BC, 2), lambda i: (i, 0)),
            ],
            out_shape=[
                jax.ShapeDtypeStruct((N, Dn), jnp.float32),
                jax.ShapeDtypeStruct((N, 2), jnp.float32),
            ],
        )(acc[0], acc[1], den[0], den[1], m, hp, aa, bp, lm,
          W, a_s, a_d)

    return run


def _make_combine_out(Dp):
    BC = 2000
    GC = N // BC

    def body(acc0_ref, acc1_ref, den0_ref, den1_ref, m_ref, hp_ref, aa_ref,
             bp_ref, lm_ref, o_ref):
        o_ref[...] = _combine_xin(Dp, acc0_ref, acc1_ref, den0_ref, den1_ref,
                                  m_ref, hp_ref, aa_ref, bp_ref, lm_ref)

    def run(acc, den, m, hp, aa, bp, lm):
        return pl.pallas_call(
            body,
            grid=(GC,),
            in_specs=[
                pl.BlockSpec((BC, Dp), lambda i: (i, 0)),
                pl.BlockSpec((BC, Dp), lambda i: (i, 0)),
                pl.BlockSpec((BC, 16), lambda i: (i, 0)),
                pl.BlockSpec((BC, 16), lambda i: (i, 0)),
                pl.BlockSpec((2, 16), lambda i: (0, 0)),
                pl.BlockSpec((BC, Dp), lambda i: (i, 0)),
                pl.BlockSpec((BC, 2), lambda i: (i, 0)),
                pl.BlockSpec((1, Dp), lambda i: (0, 0)),
                pl.BlockSpec((1, 1), lambda i: (0, 0)),
            ],
            out_specs=pl.BlockSpec((BC, Dp), lambda i: (i, 0)),
            out_shape=jax.ShapeDtypeStruct((N, Dp), jnp.float32),
        )(acc[0], acc[1], den[0], den[1], m, hp, aa, bp, lm)

    return run


_sc_alpha = _make_sc_alpha()
_KC1, _KC2, _KC3 = 80, 80, 80
_sc_scat128 = _make_sc_scatter(128, _KC1)
_sc_gat64 = _make_sc_gat(64, _KC2)
_sc_gat32 = _make_sc_gat(32, _KC3)
_comb12 = _make_combine_matmul(128, 64)
_comb23 = _make_combine_matmul(64, 32)
_comb3o = _make_combine_out(32)


def kernel(x, adj, edge_weight, W1, as1, ad1, We1, ae1, b1,
           W2, as2, ad2, We2, ae2, b2, W3, as3, ad3, We3, ae3, b3):
    adjr = adj.reshape(2, NTILES, EW)
    ewr = edge_weight.reshape(NTILES, EW)

    h1, aa1, c1v, c2v, c3v, lm1, lm2, lm3 = _dense1(
        x, W1, as1.reshape(1, -1), ad1.reshape(1, -1), edge_weight,
        We1, ae1, We2, ae2, We3, ae3)

    p1, m1 = _sc_alpha(aa1.reshape(2 * N), adjr, ewr, c1v)
    acc1, den1 = _sc_scat128(h1, adjr, p1)
    h2, aa2 = _comb12(acc1, den1, m1, h1, aa1, b1.reshape(1, -1), lm1, W2,
                      as2.reshape(1, -1), ad2.reshape(1, -1))
    acc2, den2, m2 = _sc_gat64(h2, aa2.reshape(2 * N), adjr, ewr, c2v)
    h3, aa3 = _comb23(acc2, den2, m2, h2, aa2, b2.reshape(1, -1), lm2, W3,
                      as3.reshape(1, -1), ad3.reshape(1, -1))
    acc3, den3, m3 = _sc_gat32(h3, aa3.reshape(2 * N), adjr, ewr, c3v)
    return _comb3o(acc3, den3, m3, h3, aa3, b3.reshape(1, -1), lm3)
